# baseline (speedup 1.0000x reference)
import functools

import jax
import jax.numpy as jnp
from jax import lax
from jax.experimental import pallas as pl
from jax.experimental.pallas import tpu as pltpu
from jax.experimental.pallas import tpu_sc as plsc

_NC = 2
_NS = 16
_NW = _NC * _NS
_B = 128


def _leaky(v):
    return jnp.where(v >= 0, v, 0.2 * v)


def _mm_multi_body(a_ref, w_ref, b_ref, *o_refs, widths):
    full = (
        jnp.dot(a_ref[...], w_ref[...], preferred_element_type=jnp.float32)
        + b_ref[...]
    )
    off = 0
    for o_ref, w_ in zip(o_refs, widths):
        o_ref[...] = full[:, off:off + w_]
        off += w_


def _mm_multi(a, w, b, widths, bm):
    n, k = a.shape
    m = w.shape[1]
    body = functools.partial(_mm_multi_body, widths=tuple(widths))
    return pl.pallas_call(
        body,
        grid=(n // bm,),
        in_specs=[
            pl.BlockSpec((bm, k), lambda i: (i, 0)),
            pl.BlockSpec((k, m), lambda i: (0, 0)),
            pl.BlockSpec((1, m), lambda i: (0, 0)),
        ],
        out_specs=[pl.BlockSpec((bm, w_), lambda i: (i, 0)) for w_ in widths],
        out_shape=[jax.ShapeDtypeStruct((n, w_), jnp.float32)
                   for w_ in widths],
    )(a, w, b.reshape(1, m))


def _aggr_update(refs, aggr_w, cnt_from_s, n_cols):
    sparts = refs[:2 * n_cols]
    refs = refs[2 * n_cols:]
    msums = [sparts[k][...] + sparts[n_cols + k][...] for k in range(n_cols)]
    if cnt_from_s:
        cnt = jnp.maximum(msums[0][:, aggr_w:aggr_w + 1], 1.0)
        aggr = msums[0][:, :aggr_w]
    else:
        c0_ref, c1_ref = refs[:2]
        refs = refs[2:]
        cnt = jnp.maximum(c0_ref[...][:, 0:1] + c1_ref[...][:, 0:1], 1.0)
        aggr = msums[0] if n_cols == 1 else jnp.concatenate(msums, axis=1)
    aggr = aggr / cnt
    hu_ref, res_ref, wu_ref, g_ref, b_ref = refs[:5]
    u = (
        jnp.dot(aggr, wu_ref[...], preferred_element_type=jnp.float32)
        + hu_ref[...]
    )
    hn = _leaky(u) + res_ref[...]
    mu = jnp.mean(hn, axis=-1, keepdims=True)
    var = jnp.mean((hn - mu) ** 2, axis=-1, keepdims=True)
    h = (hn - mu) * lax.rsqrt(var + 1e-5) * g_ref[...] + b_ref[...]
    return h, refs[5:]


def _upd_mm_body(*refs, aggr_w, cnt_from_s, n_cols, widths):
    n_out = len(widths)
    h, rest = _aggr_update(refs[:len(refs) - n_out], aggr_w, cnt_from_s,
                           n_cols)
    cw_ref, cb_ref = rest
    _mm_multi_body_from(h, cw_ref, cb_ref, refs[len(refs) - n_out:], widths)


def _mm_multi_body_from(a, w_ref, b_ref, o_refs, widths):
    full = (
        jnp.dot(a, w_ref[...], preferred_element_type=jnp.float32)
        + b_ref[...]
    )
    off = 0
    for o_ref, w_ in zip(o_refs, widths):
        o_ref[...] = full[:, off:off + w_]
        off += w_


def _upd_mm(sparts, cc, hu, res, wu, g, b, cat_w, cat_b, widths,
            aggr_w, bm):
    n, h = hu.shape
    n_cols = len(sparts) // 2
    cnt_from_s = cc is None
    outw = cat_w.shape[1]
    body = functools.partial(
        _upd_mm_body, aggr_w=aggr_w, cnt_from_s=cnt_from_s, n_cols=n_cols,
        widths=tuple(widths))
    blk = lambda w_: pl.BlockSpec((bm, w_), lambda i: (i, 0))
    rep = lambda r_, w_: pl.BlockSpec((r_, w_), lambda i: (0, 0))
    in_specs = [blk(s.shape[1]) for s in sparts]
    args = list(sparts)
    if not cnt_from_s:
        in_specs += [blk(16), blk(16)]
        args += list(cc)
    in_specs += [blk(h), blk(h), rep(aggr_w, h), rep(1, h), rep(1, h),
                 rep(h, outw), rep(1, outw)]
    args += [hu, res, wu, g.reshape(1, h), b.reshape(1, h), cat_w,
             cat_b.reshape(1, outw)]
    return pl.pallas_call(
        body,
        grid=(n // bm,),
        in_specs=in_specs,
        out_specs=[pl.BlockSpec((bm, w_), lambda i: (i, 0)) for w_ in widths],
        out_shape=[jax.ShapeDtypeStruct((n, w_), jnp.float32)
                   for w_ in widths],
    )(*args)


def _upd_head_body(*refs, aggr_w, n_cols, num_graphs, last):
    i = pl.program_id(0)
    gs_acc, cnt_acc = refs[-2:]
    o_ref = refs[-3]
    h, rest = _aggr_update(refs[:len(refs) - 10], aggr_w, False, n_cols)
    b_ref, w0_ref, b0_ref, w1_ref, b1_ref, w2_ref, b2_ref = refs[
        len(refs) - 10:len(refs) - 3]

    @pl.when(i == 0)
    def _():
        gs_acc[...] = jnp.zeros_like(gs_acc)
        cnt_acc[...] = jnp.zeros_like(cnt_acc)

    bid = b_ref[...]
    gi = lax.broadcasted_iota(jnp.int32, (1, num_graphs), 1)
    oh = (bid == gi).astype(jnp.float32)
    dn = (((0,), (0,)), ((), ()))
    gs_acc[...] += lax.dot_general(oh, h, dn,
                                   preferred_element_type=jnp.float32)
    ones = jnp.ones((oh.shape[0], 8), jnp.float32)
    cnt_acc[...] += lax.dot_general(oh, ones, dn,
                                    preferred_element_type=jnp.float32)

    @pl.when(i == last)
    def _():
        ge = gs_acc[...] / jnp.maximum(cnt_acc[...][:, 0:1], 1.0)
        z = _leaky(jnp.dot(ge, w0_ref[...],
                           preferred_element_type=jnp.float32) + b0_ref[...])
        z = _leaky(jnp.dot(z, w1_ref[...],
                           preferred_element_type=jnp.float32) + b1_ref[...])
        o_ref[...] = (
            jnp.dot(z, w2_ref[...], preferred_element_type=jnp.float32)
            + b2_ref[...]
        )


def _upd_head(sparts, cc, hu, res, wu, g, b, batch2d, mp, num_graphs, bm):
    n, h = hu.shape
    n_cols = len(sparts) // 2
    grid = n // bm
    w2p = jnp.zeros((64, 128), jnp.float32).at[:, 0:1].set(mp["W2"])
    b2p = jnp.zeros((128,), jnp.float32).at[0].set(mp["b2"][0])
    body = functools.partial(
        _upd_head_body, aggr_w=h, n_cols=n_cols, num_graphs=num_graphs,
        last=grid - 1)
    blk = lambda w_: pl.BlockSpec((bm, w_), lambda i: (i, 0))
    rep = lambda r_, w_: pl.BlockSpec((r_, w_), lambda i: (0, 0))
    in_specs = [blk(s.shape[1]) for s in sparts]
    in_specs += [blk(16), blk(16), blk(h), blk(h), rep(h, h), rep(1, h),
                 rep(1, h), blk(1), rep(h, 128), rep(1, 128), rep(128, 64),
                 rep(1, 64), rep(64, 128), rep(1, 128)]
    out = pl.pallas_call(
        body,
        grid=(grid,),
        in_specs=in_specs,
        out_specs=pl.BlockSpec((num_graphs, 128), lambda i: (0, 0)),
        out_shape=jax.ShapeDtypeStruct((num_graphs, 128), jnp.float32),
        scratch_shapes=[
            pltpu.VMEM((num_graphs, h), jnp.float32),
            pltpu.VMEM((num_graphs, 8), jnp.float32),
        ],
    )(*sparts, *cc, hu, res, wu, g.reshape(1, h), b.reshape(1, h), batch2d,
      mp["W0"], mp["b0"].reshape(1, 128), mp["W1"], mp["b1"].reshape(1, 64),
      w2p, b2p.reshape(1, 128))
    return out[:, 0:1]


def _sc_edge_body(tab_ref, ew_ref, src_ref, dst_ref, zer_ref,
                  out0_ref, out1_ref,
                  isrc, idst, rows0, rows1, rows2, shared,
                  ew_sems, g_sems, sc_sems,
                  *, width, comp_w, n_chunks, max_cw, rows_per_tile):
    cid = lax.axis_index("c")
    sid = lax.axis_index("s")
    wid = sid * _NC + cid
    rows_bufs = (rows0, rows1, rows2)

    pltpu.sync_copy(zer_ref,
                    shared.at[pl.ds(sid * rows_per_tile, rows_per_tile)])

    lo = (n_chunks * wid) // _NW
    hi = (n_chunks * (wid + 1)) // _NW
    cw = hi - lo
    plsc.subcore_barrier()

    def ew_start(t, s):
        @pl.when(t < cw)
        def _():
            @pl.when(t >= 3)
            def _():
                pltpu.make_async_copy(rows_bufs[s], shared.at[idst.at[s]],
                                      sc_sems[s]).wait()
            base = (lo + t) * _B
            pltpu.async_copy(src_ref.at[pl.ds(base, _B)], isrc.at[s],
                             ew_sems[s])
            pltpu.async_copy(dst_ref.at[pl.ds(base, _B)], idst.at[s],
                             ew_sems[s])
            pltpu.async_copy(ew_ref.at[pl.ds(base, _B)], rows_bufs[s],
                             ew_sems[s])

    def gather_start(t, s):
        @pl.when(t < cw)
        def _():
            base = (lo + t) * _B
            pltpu.make_async_copy(src_ref.at[pl.ds(base, _B)], isrc.at[s],
                                  ew_sems[s]).wait()
            pltpu.make_async_copy(dst_ref.at[pl.ds(base, _B)], idst.at[s],
                                  ew_sems[s]).wait()
            pltpu.make_async_copy(ew_ref.at[pl.ds(base, _B)], rows_bufs[s],
                                  ew_sems[s]).wait()
            pltpu.async_copy(tab_ref.at[isrc.at[s]], rows_bufs[s],
                             g_sems[s], add=True)

    def compute(t, s):
        @pl.when(jnp.logical_and(t >= 0, t < cw))
        def _():
            pltpu.make_async_copy(tab_ref.at[isrc.at[s]], rows_bufs[s],
                                  g_sems[s]).wait()
            buf = rows_bufs[s]

            def row_body(r, cc):
                for col in range(comp_w // 16):
                    sl = pl.ds(col * 16, 16)
                    t0 = buf[r, sl]
                    buf[r, sl] = jnp.maximum(t0, 0.2 * t0)
                return cc

            lax.fori_loop(0, _B, row_body, 0, unroll=4)
            pltpu.async_copy(buf, shared.at[idst.at[s]], sc_sems[s],
                             add=True)

    ew_start(jnp.int32(0), 0)

    def group_body(u, carry):
        for si in range(3):
            t = 3 * u + si
            gather_start(t, si)
            ew_start(t + 1, (si + 1) % 3)
            compute(t - 1, (si + 2) % 3)
        return carry

    n_groups = (max_cw + 1 + 2) // 3
    lax.fori_loop(0, n_groups, group_body, 0)

    for s in range(3):
        pltpu.make_async_copy(rows_bufs[s], shared.at[idst.at[s]],
                              sc_sems[s]).wait()
    plsc.subcore_barrier()

    row0 = sid * rows_per_tile

    @pl.when(cid == 0)
    def _():
        pltpu.sync_copy(shared.at[pl.ds(row0, rows_per_tile)],
                        out0_ref.at[pl.ds(row0, rows_per_tile)])

    @pl.when(cid == 1)
    def _():
        pltpu.sync_copy(shared.at[pl.ds(row0, rows_per_tile)],
                        out1_ref.at[pl.ds(row0, rows_per_tile)])


def _sc_edge(tab, ew, src, dst, comp_w=None):
    n_nodes, width = tab.shape
    comp_w = width if comp_w is None else comp_w
    n_edges = src.shape[0]
    n_chunks = n_edges // _B
    max_cw = -(-n_chunks // _NW)
    rows_per_tile = -(-n_nodes // (8 * _NS)) * 8
    n_pad = rows_per_tile * _NS
    mesh = plsc.VectorSubcoreMesh(core_axis_name="c", subcore_axis_name="s",
                                  num_cores=_NC, num_subcores=_NS)
    body = functools.partial(
        _sc_edge_body, width=width, comp_w=comp_w, n_chunks=n_chunks,
        max_cw=max_cw, rows_per_tile=rows_per_tile)
    zer = jnp.zeros((rows_per_tile, width), jnp.float32)
    s0, s1 = pl.kernel(
        body,
        out_type=(jax.ShapeDtypeStruct((n_pad, width), jnp.float32),) * 2,
        mesh=mesh,
        scratch_types=[
            pltpu.VMEM((3, _B), jnp.int32),
            pltpu.VMEM((3, _B), jnp.int32),
            pltpu.VMEM((_B, width), jnp.float32),
            pltpu.VMEM((_B, width), jnp.float32),
            pltpu.VMEM((_B, width), jnp.float32),
            pltpu.VMEM_SHARED((n_pad, width), jnp.float32),
            [pltpu.SemaphoreType.DMA] * 3,
            [pltpu.SemaphoreType.DMA] * 3,
            [pltpu.SemaphoreType.DMA] * 3,
        ],
    )(tab, ew, src, dst, zer)
    return s0, s1


def _layer_weights(p, din, hdim, de):
    w_x = p["msg_W"][:din]
    w_e = p["msg_W"][din:]
    if hdim < 128:
        padw = 128 - hdim
        tabw = 128
        tab_w = jnp.concatenate([w_x, jnp.zeros((din, padw))], axis=1)
        tab_b = jnp.zeros((tabw,), jnp.float32).at[hdim].set(1.0)
        ew_w = jnp.concatenate([w_e, jnp.zeros((de, padw))], axis=1)
        ew_b = jnp.concatenate([p["msg_b"], jnp.zeros((padw,), jnp.float32)])
    else:
        tabw = hdim
        tab_w = w_x
        tab_b = jnp.zeros((tabw,), jnp.float32)
        ew_w = w_e
        ew_b = p["msg_b"]
    cat_w = jnp.concatenate([tab_w, p["upd_W"][hdim:], p["res_W"]], axis=1)
    cat_b = jnp.concatenate([tab_b, p["upd_b"], p["res_b"]])
    tab_widths = [128] * (tabw // 128)
    return cat_w, cat_b, ew_w, ew_b, tab_widths


def kernel(x, edge_index, edge_attr, batch, params):
    src = edge_index[0]
    dst = edge_index[1]
    n, dn = x.shape
    de = edge_attr.shape[1]
    hid = (64, 128, 256)
    bm_n = 400
    p0, p1, p2 = (params[f"layer{i}"] for i in range(3))

    cw0, cb0, eww0, ewb0, tw0 = _layer_weights(p0, dn, hid[0], de)
    cw1, cb1, eww1, ewb1, tw1 = _layer_weights(p1, hid[0], hid[1], de)
    cw2, cb2, eww2, ewb2, tw2 = _layer_weights(p2, hid[1], hid[2], de)

    (ew0,) = _mm_multi(edge_attr, eww0, ewb0, [128], 1280)
    (ew1,) = _mm_multi(edge_attr, eww1, ewb1, [128], 1280)
    ew2a, ew2b = _mm_multi(edge_attr, eww2, ewb2, [128, 128], 1280)

    tab0, hu0, res0 = _mm_multi(x, cw0, cb0, [128, hid[0], hid[0]], bm_n)
    s0_0, s1_0 = _sc_edge(tab0, ew0, src, dst, comp_w=hid[0] + 16)
    c0 = s0_0[:, hid[0]:hid[0] + 16]
    c1 = s1_0[:, hid[0]:hid[0] + 16]

    tab1, hu1, res1 = _upd_mm(
        [s0_0, s1_0], None, hu0, res0, p0["upd_W"][:hid[0]], p0["ln_g"],
        p0["ln_b"], cw1, cb1, [128, hid[1], hid[1]], hid[0], bm_n)

    s0_1, s1_1 = _sc_edge(tab1, ew1, src, dst)

    tab2a, tab2b, hu2, res2 = _upd_mm(
        [s0_1, s1_1], (c0, c1), hu1, res1, p1["upd_W"][:hid[1]], p1["ln_g"],
        p1["ln_b"], cw2, cb2, [128, 128, hid[2], hid[2]], hid[1], bm_n)

    s0_2a, s1_2a = _sc_edge(tab2a, ew2a, src, dst)
    s0_2b, s1_2b = _sc_edge(tab2b, ew2b, src, dst)

    batch2d = batch.reshape(n, 1)
    return _upd_head(
        [s0_2a, s0_2b, s1_2a, s1_2b], (c0, c1), hu2, res2,
        p2["upd_W"][:hid[2]], p2["ln_g"], p2["ln_b"], batch2d,
        params["mlp"], 64, bm_n)


# speedup vs baseline: 1.0475x; 1.0475x over previous
import functools

import jax
import jax.numpy as jnp
from jax import lax
from jax.experimental import pallas as pl
from jax.experimental.pallas import tpu as pltpu
from jax.experimental.pallas import tpu_sc as plsc

_NC = 2
_NS = 16
_NW = _NC * _NS
_B = 128


def _leaky(v):
    return jnp.where(v >= 0, v, 0.2 * v)


def _mm_multi_body(a_ref, w_ref, b_ref, *o_refs, widths):
    full = (
        jnp.dot(a_ref[...], w_ref[...], preferred_element_type=jnp.float32)
        + b_ref[...]
    )
    off = 0
    for o_ref, w_ in zip(o_refs, widths):
        o_ref[...] = full[:, off:off + w_]
        off += w_


def _mm_multi(a, w, b, widths, bm):
    n, k = a.shape
    m = w.shape[1]
    body = functools.partial(_mm_multi_body, widths=tuple(widths))
    return pl.pallas_call(
        body,
        grid=(n // bm,),
        in_specs=[
            pl.BlockSpec((bm, k), lambda i: (i, 0)),
            pl.BlockSpec((k, m), lambda i: (0, 0)),
            pl.BlockSpec((1, m), lambda i: (0, 0)),
        ],
        out_specs=[pl.BlockSpec((bm, w_), lambda i: (i, 0)) for w_ in widths],
        out_shape=[jax.ShapeDtypeStruct((n, w_), jnp.float32)
                   for w_ in widths],
    )(a, w, b.reshape(1, m))


def _aggr_update(refs, aggr_w, cnt_from_s, n_cols):
    sparts = refs[:2 * n_cols]
    refs = refs[2 * n_cols:]
    msums = [sparts[k][...] + sparts[n_cols + k][...] for k in range(n_cols)]
    if cnt_from_s:
        cnt = jnp.maximum(msums[0][:, aggr_w:aggr_w + 1], 1.0)
        aggr = msums[0][:, :aggr_w]
    else:
        c0_ref, c1_ref = refs[:2]
        refs = refs[2:]
        cnt = jnp.maximum(c0_ref[...][:, 0:1] + c1_ref[...][:, 0:1], 1.0)
        aggr = msums[0] if n_cols == 1 else jnp.concatenate(msums, axis=1)
    aggr = aggr / cnt
    hu_ref, res_ref, wu_ref, g_ref, b_ref = refs[:5]
    u = (
        jnp.dot(aggr, wu_ref[...], preferred_element_type=jnp.float32)
        + hu_ref[...]
    )
    hn = _leaky(u) + res_ref[...]
    mu = jnp.mean(hn, axis=-1, keepdims=True)
    var = jnp.mean((hn - mu) ** 2, axis=-1, keepdims=True)
    h = (hn - mu) * lax.rsqrt(var + 1e-5) * g_ref[...] + b_ref[...]
    return h, refs[5:]


def _upd_mm_body(*refs, aggr_w, cnt_from_s, n_cols, widths):
    n_out = len(widths)
    h, rest = _aggr_update(refs[:len(refs) - n_out], aggr_w, cnt_from_s,
                           n_cols)
    cw_ref, cb_ref = rest
    _mm_multi_body_from(h, cw_ref, cb_ref, refs[len(refs) - n_out:], widths)


def _mm_multi_body_from(a, w_ref, b_ref, o_refs, widths):
    full = (
        jnp.dot(a, w_ref[...], preferred_element_type=jnp.float32)
        + b_ref[...]
    )
    off = 0
    for o_ref, w_ in zip(o_refs, widths):
        o_ref[...] = full[:, off:off + w_]
        off += w_


def _upd_mm(sparts, cc, hu, res, wu, g, b, cat_w, cat_b, widths,
            aggr_w, bm):
    n, h = hu.shape
    n_cols = len(sparts) // 2
    cnt_from_s = cc is None
    outw = cat_w.shape[1]
    body = functools.partial(
        _upd_mm_body, aggr_w=aggr_w, cnt_from_s=cnt_from_s, n_cols=n_cols,
        widths=tuple(widths))
    blk = lambda w_: pl.BlockSpec((bm, w_), lambda i: (i, 0))
    rep = lambda r_, w_: pl.BlockSpec((r_, w_), lambda i: (0, 0))
    in_specs = [blk(s.shape[1]) for s in sparts]
    args = list(sparts)
    if not cnt_from_s:
        in_specs += [blk(16), blk(16)]
        args += list(cc)
    in_specs += [blk(h), blk(h), rep(aggr_w, h), rep(1, h), rep(1, h),
                 rep(h, outw), rep(1, outw)]
    args += [hu, res, wu, g.reshape(1, h), b.reshape(1, h), cat_w,
             cat_b.reshape(1, outw)]
    return pl.pallas_call(
        body,
        grid=(n // bm,),
        in_specs=in_specs,
        out_specs=[pl.BlockSpec((bm, w_), lambda i: (i, 0)) for w_ in widths],
        out_shape=[jax.ShapeDtypeStruct((n, w_), jnp.float32)
                   for w_ in widths],
    )(*args)


def _upd_head_body(*refs, aggr_w, n_cols, num_graphs, last):
    i = pl.program_id(0)
    gs_acc, cnt_acc = refs[-2:]
    o_ref = refs[-3]
    h, rest = _aggr_update(refs[:len(refs) - 10], aggr_w, False, n_cols)
    b_ref, w0_ref, b0_ref, w1_ref, b1_ref, w2_ref, b2_ref = refs[
        len(refs) - 10:len(refs) - 3]

    @pl.when(i == 0)
    def _():
        gs_acc[...] = jnp.zeros_like(gs_acc)
        cnt_acc[...] = jnp.zeros_like(cnt_acc)

    bid = b_ref[...]
    gi = lax.broadcasted_iota(jnp.int32, (1, num_graphs), 1)
    oh = (bid == gi).astype(jnp.float32)
    dn = (((0,), (0,)), ((), ()))
    gs_acc[...] += lax.dot_general(oh, h, dn,
                                   preferred_element_type=jnp.float32)
    ones = jnp.ones((oh.shape[0], 8), jnp.float32)
    cnt_acc[...] += lax.dot_general(oh, ones, dn,
                                    preferred_element_type=jnp.float32)

    @pl.when(i == last)
    def _():
        ge = gs_acc[...] / jnp.maximum(cnt_acc[...][:, 0:1], 1.0)
        z = _leaky(jnp.dot(ge, w0_ref[...],
                           preferred_element_type=jnp.float32) + b0_ref[...])
        z = _leaky(jnp.dot(z, w1_ref[...],
                           preferred_element_type=jnp.float32) + b1_ref[...])
        o_ref[...] = (
            jnp.dot(z, w2_ref[...], preferred_element_type=jnp.float32)
            + b2_ref[...]
        )


def _upd_head(sparts, cc, hu, res, wu, g, b, batch2d, mp, num_graphs, bm):
    n, h = hu.shape
    n_cols = len(sparts) // 2
    grid = n // bm
    w2p = jnp.zeros((64, 128), jnp.float32).at[:, 0:1].set(mp["W2"])
    b2p = jnp.zeros((128,), jnp.float32).at[0].set(mp["b2"][0])
    body = functools.partial(
        _upd_head_body, aggr_w=h, n_cols=n_cols, num_graphs=num_graphs,
        last=grid - 1)
    blk = lambda w_: pl.BlockSpec((bm, w_), lambda i: (i, 0))
    rep = lambda r_, w_: pl.BlockSpec((r_, w_), lambda i: (0, 0))
    in_specs = [blk(s.shape[1]) for s in sparts]
    in_specs += [blk(16), blk(16), blk(h), blk(h), rep(h, h), rep(1, h),
                 rep(1, h), blk(1), rep(h, 128), rep(1, 128), rep(128, 64),
                 rep(1, 64), rep(64, 128), rep(1, 128)]
    out = pl.pallas_call(
        body,
        grid=(grid,),
        in_specs=in_specs,
        out_specs=pl.BlockSpec((num_graphs, 128), lambda i: (0, 0)),
        out_shape=jax.ShapeDtypeStruct((num_graphs, 128), jnp.float32),
        scratch_shapes=[
            pltpu.VMEM((num_graphs, h), jnp.float32),
            pltpu.VMEM((num_graphs, 8), jnp.float32),
        ],
    )(*sparts, *cc, hu, res, wu, g.reshape(1, h), b.reshape(1, h), batch2d,
      mp["W0"], mp["b0"].reshape(1, 128), mp["W1"], mp["b1"].reshape(1, 64),
      w2p, b2p.reshape(1, 128))
    return out[:, 0:1]


def _sc_edge_body(tab_ref, ew_ref, src_ref, dst_ref, zer_ref,
                  out0_ref, out1_ref,
                  isrc, idst, rows0, rows1, rows2, shared,
                  ew_sems, g_sems, sc_sems,
                  *, width, comp_w, n_chunks, max_cw, rows_per_tile):
    cid = lax.axis_index("c")
    sid = lax.axis_index("s")
    wid = sid * _NC + cid
    rows_bufs = (rows0, rows1, rows2)

    pltpu.sync_copy(zer_ref,
                    shared.at[pl.ds(sid * rows_per_tile, rows_per_tile)])

    lo = (n_chunks * wid) // _NW
    hi = (n_chunks * (wid + 1)) // _NW
    cw = hi - lo
    plsc.subcore_barrier()

    def ew_start(t, s):
        @pl.when(t < cw)
        def _():
            @pl.when(t >= 3)
            def _():
                pltpu.make_async_copy(rows_bufs[s], shared.at[idst.at[s]],
                                      sc_sems[s]).wait()
            base = (lo + t) * _B
            pltpu.async_copy(src_ref.at[pl.ds(base, _B)], isrc.at[s],
                             ew_sems[s])
            pltpu.async_copy(dst_ref.at[pl.ds(base, _B)], idst.at[s],
                             ew_sems[s])
            pltpu.async_copy(ew_ref.at[pl.ds(base, _B)], rows_bufs[s],
                             ew_sems[s])

    def gather_start(t, s):
        @pl.when(t < cw)
        def _():
            base = (lo + t) * _B
            pltpu.make_async_copy(src_ref.at[pl.ds(base, _B)], isrc.at[s],
                                  ew_sems[s]).wait()
            pltpu.make_async_copy(dst_ref.at[pl.ds(base, _B)], idst.at[s],
                                  ew_sems[s]).wait()
            pltpu.make_async_copy(ew_ref.at[pl.ds(base, _B)], rows_bufs[s],
                                  ew_sems[s]).wait()
            pltpu.async_copy(tab_ref.at[isrc.at[s]], rows_bufs[s],
                             g_sems[s], add=True)

    def compute(t, s):
        @pl.when(jnp.logical_and(t >= 0, t < cw))
        def _():
            pltpu.make_async_copy(tab_ref.at[isrc.at[s]], rows_bufs[s],
                                  g_sems[s]).wait()
            buf = rows_bufs[s]

            def row_body(r, cc):
                for col in range(comp_w // 16):
                    sl = pl.ds(col * 16, 16)
                    t0 = buf[r, sl]
                    buf[r, sl] = jnp.maximum(t0, 0.2 * t0)
                return cc

            lax.fori_loop(0, _B, row_body, 0, unroll=2)
            pltpu.async_copy(buf, shared.at[idst.at[s]], sc_sems[s],
                             add=True)

    ew_start(jnp.int32(0), 0)

    def group_body(u, carry):
        for si in range(3):
            t = 3 * u + si
            gather_start(t, si)
            ew_start(t + 1, (si + 1) % 3)
            compute(t - 1, (si + 2) % 3)
        return carry

    n_groups = (max_cw + 1 + 2) // 3
    lax.fori_loop(0, n_groups, group_body, 0)

    for s in range(3):
        pltpu.make_async_copy(rows_bufs[s], shared.at[idst.at[s]],
                              sc_sems[s]).wait()
    plsc.subcore_barrier()

    row0 = sid * rows_per_tile

    @pl.when(cid == 0)
    def _():
        pltpu.sync_copy(shared.at[pl.ds(row0, rows_per_tile)],
                        out0_ref.at[pl.ds(row0, rows_per_tile)])

    @pl.when(cid == 1)
    def _():
        pltpu.sync_copy(shared.at[pl.ds(row0, rows_per_tile)],
                        out1_ref.at[pl.ds(row0, rows_per_tile)])


def _sc_edge(tab, ew, src, dst, comp_w=None):
    n_nodes, width = tab.shape
    comp_w = width if comp_w is None else comp_w
    n_edges = src.shape[0]
    n_chunks = n_edges // _B
    max_cw = -(-n_chunks // _NW)
    rows_per_tile = -(-n_nodes // (8 * _NS)) * 8
    n_pad = rows_per_tile * _NS
    mesh = plsc.VectorSubcoreMesh(core_axis_name="c", subcore_axis_name="s",
                                  num_cores=_NC, num_subcores=_NS)
    body = functools.partial(
        _sc_edge_body, width=width, comp_w=comp_w, n_chunks=n_chunks,
        max_cw=max_cw, rows_per_tile=rows_per_tile)
    zer = jnp.zeros((rows_per_tile, width), jnp.float32)
    s0, s1 = pl.kernel(
        body,
        out_type=(jax.ShapeDtypeStruct((n_pad, width), jnp.float32),) * 2,
        mesh=mesh,
        scratch_types=[
            pltpu.VMEM((3, _B), jnp.int32),
            pltpu.VMEM((3, _B), jnp.int32),
            pltpu.VMEM((_B, width), jnp.float32),
            pltpu.VMEM((_B, width), jnp.float32),
            pltpu.VMEM((_B, width), jnp.float32),
            pltpu.VMEM_SHARED((n_pad, width), jnp.float32),
            [pltpu.SemaphoreType.DMA] * 3,
            [pltpu.SemaphoreType.DMA] * 3,
            [pltpu.SemaphoreType.DMA] * 3,
        ],
    )(tab, ew, src, dst, zer)
    return s0, s1


def _layer_weights(p, din, hdim, de):
    w_x = p["msg_W"][:din]
    w_e = p["msg_W"][din:]
    if hdim < 128:
        padw = 128 - hdim
        tabw = 128
        tab_w = jnp.concatenate([w_x, jnp.zeros((din, padw))], axis=1)
        tab_b = jnp.zeros((tabw,), jnp.float32).at[hdim].set(1.0)
        ew_w = jnp.concatenate([w_e, jnp.zeros((de, padw))], axis=1)
        ew_b = jnp.concatenate([p["msg_b"], jnp.zeros((padw,), jnp.float32)])
    else:
        tabw = hdim
        tab_w = w_x
        tab_b = jnp.zeros((tabw,), jnp.float32)
        ew_w = w_e
        ew_b = p["msg_b"]
    cat_w = jnp.concatenate([tab_w, p["upd_W"][hdim:], p["res_W"]], axis=1)
    cat_b = jnp.concatenate([tab_b, p["upd_b"], p["res_b"]])
    tab_widths = [128] * (tabw // 128)
    return cat_w, cat_b, ew_w, ew_b, tab_widths


def kernel(x, edge_index, edge_attr, batch, params):
    src = edge_index[0]
    dst = edge_index[1]
    n, dn = x.shape
    de = edge_attr.shape[1]
    hid = (64, 128, 256)
    bm_n = 400
    p0, p1, p2 = (params[f"layer{i}"] for i in range(3))

    cw0, cb0, eww0, ewb0, tw0 = _layer_weights(p0, dn, hid[0], de)
    cw1, cb1, eww1, ewb1, tw1 = _layer_weights(p1, hid[0], hid[1], de)
    cw2, cb2, eww2, ewb2, tw2 = _layer_weights(p2, hid[1], hid[2], de)

    ew_w = jnp.concatenate([eww0, eww1, eww2], axis=1)
    ew_b = jnp.concatenate([ewb0, ewb1, ewb2])
    ew0, ew1, ew2a, ew2b = _mm_multi(edge_attr, ew_w, ew_b,
                                     [128, 128, 128, 128], 1280)

    tab0, hu0, res0 = _mm_multi(x, cw0, cb0, [128, hid[0], hid[0]], bm_n)
    s0_0, s1_0 = _sc_edge(tab0, ew0, src, dst, comp_w=hid[0] + 16)
    c0 = s0_0[:, hid[0]:hid[0] + 16]
    c1 = s1_0[:, hid[0]:hid[0] + 16]

    tab1, hu1, res1 = _upd_mm(
        [s0_0, s1_0], None, hu0, res0, p0["upd_W"][:hid[0]], p0["ln_g"],
        p0["ln_b"], cw1, cb1, [128, hid[1], hid[1]], hid[0], bm_n)

    s0_1, s1_1 = _sc_edge(tab1, ew1, src, dst)

    tab2a, tab2b, hu2, res2 = _upd_mm(
        [s0_1, s1_1], (c0, c1), hu1, res1, p1["upd_W"][:hid[1]], p1["ln_g"],
        p1["ln_b"], cw2, cb2, [128, 128, hid[2], hid[2]], hid[1], bm_n)

    s0_2a, s1_2a = _sc_edge(tab2a, ew2a, src, dst)
    s0_2b, s1_2b = _sc_edge(tab2b, ew2b, src, dst)

    batch2d = batch.reshape(n, 1)
    return _upd_head(
        [s0_2a, s0_2b, s1_2a, s1_2b], (c0, c1), hu2, res2,
        p2["upd_W"][:hid[2]], p2["ln_g"], p2["ln_b"], batch2d,
        params["mlp"], 64, bm_n)


# speedup vs baseline: 1.1087x; 1.0584x over previous
import functools

import jax
import jax.numpy as jnp
from jax import lax
from jax.experimental import pallas as pl
from jax.experimental.pallas import tpu as pltpu
from jax.experimental.pallas import tpu_sc as plsc

_NC = 2
_NS = 16
_NW = _NC * _NS
_B = 128


def _leaky(v):
    return jnp.where(v >= 0, v, 0.2 * v)


def _mm_multi_body(a_ref, w_ref, b_ref, *o_refs, widths):
    full = (
        jnp.dot(a_ref[...], w_ref[...], preferred_element_type=jnp.float32)
        + b_ref[...]
    )
    off = 0
    for o_ref, w_ in zip(o_refs, widths):
        o_ref[...] = full[:, off:off + w_]
        off += w_


def _mm_multi(a, w, b, widths, bm):
    n, k = a.shape
    m = w.shape[1]
    body = functools.partial(_mm_multi_body, widths=tuple(widths))
    return pl.pallas_call(
        body,
        grid=(n // bm,),
        in_specs=[
            pl.BlockSpec((bm, k), lambda i: (i, 0)),
            pl.BlockSpec((k, m), lambda i: (0, 0)),
            pl.BlockSpec((1, m), lambda i: (0, 0)),
        ],
        out_specs=[pl.BlockSpec((bm, w_), lambda i: (i, 0)) for w_ in widths],
        out_shape=[jax.ShapeDtypeStruct((n, w_), jnp.float32)
                   for w_ in widths],
    )(a, w, b.reshape(1, m))


def _aggr_update(refs, aggr_w, cnt_from_s, n_cols):
    sparts = refs[:2 * n_cols]
    refs = refs[2 * n_cols:]
    msums = [sparts[k][...] + sparts[n_cols + k][...] for k in range(n_cols)]
    if cnt_from_s:
        cnt = jnp.maximum(msums[0][:, aggr_w:aggr_w + 1], 1.0)
        aggr = msums[0][:, :aggr_w]
    else:
        c0_ref, c1_ref = refs[:2]
        refs = refs[2:]
        cnt = jnp.maximum(c0_ref[...][:, 0:1] + c1_ref[...][:, 0:1], 1.0)
        aggr = msums[0] if n_cols == 1 else jnp.concatenate(msums, axis=1)
    aggr = aggr / cnt
    hu_ref, res_ref, wu_ref, g_ref, b_ref = refs[:5]
    u = (
        jnp.dot(aggr, wu_ref[...], preferred_element_type=jnp.float32)
        + hu_ref[...]
    )
    hn = _leaky(u) + res_ref[...]
    mu = jnp.mean(hn, axis=-1, keepdims=True)
    var = jnp.mean((hn - mu) ** 2, axis=-1, keepdims=True)
    h = (hn - mu) * lax.rsqrt(var + 1e-5) * g_ref[...] + b_ref[...]
    return h, refs[5:]


def _upd_mm_body(*refs, aggr_w, cnt_from_s, n_cols, widths):
    n_out = len(widths)
    h, rest = _aggr_update(refs[:len(refs) - n_out], aggr_w, cnt_from_s,
                           n_cols)
    cw_ref, cb_ref = rest
    _mm_multi_body_from(h, cw_ref, cb_ref, refs[len(refs) - n_out:], widths)


def _mm_multi_body_from(a, w_ref, b_ref, o_refs, widths):
    full = (
        jnp.dot(a, w_ref[...], preferred_element_type=jnp.float32)
        + b_ref[...]
    )
    off = 0
    for o_ref, w_ in zip(o_refs, widths):
        o_ref[...] = full[:, off:off + w_]
        off += w_


def _upd_mm(sparts, cc, hu, res, wu, g, b, cat_w, cat_b, widths,
            aggr_w, bm):
    n, h = hu.shape
    n_cols = len(sparts) // 2
    cnt_from_s = cc is None
    outw = cat_w.shape[1]
    body = functools.partial(
        _upd_mm_body, aggr_w=aggr_w, cnt_from_s=cnt_from_s, n_cols=n_cols,
        widths=tuple(widths))
    blk = lambda w_: pl.BlockSpec((bm, w_), lambda i: (i, 0))
    rep = lambda r_, w_: pl.BlockSpec((r_, w_), lambda i: (0, 0))
    in_specs = [blk(s.shape[1]) for s in sparts]
    args = list(sparts)
    if not cnt_from_s:
        in_specs += [blk(16), blk(16)]
        args += list(cc)
    in_specs += [blk(h), blk(h), rep(aggr_w, h), rep(1, h), rep(1, h),
                 rep(h, outw), rep(1, outw)]
    args += [hu, res, wu, g.reshape(1, h), b.reshape(1, h), cat_w,
             cat_b.reshape(1, outw)]
    return pl.pallas_call(
        body,
        grid=(n // bm,),
        in_specs=in_specs,
        out_specs=[pl.BlockSpec((bm, w_), lambda i: (i, 0)) for w_ in widths],
        out_shape=[jax.ShapeDtypeStruct((n, w_), jnp.float32)
                   for w_ in widths],
    )(*args)


def _upd_head_body(*refs, aggr_w, n_cols, num_graphs, last):
    i = pl.program_id(0)
    gs_acc, cnt_acc = refs[-2:]
    o_ref = refs[-3]
    h, rest = _aggr_update(refs[:len(refs) - 10], aggr_w, False, n_cols)
    b_ref, w0_ref, b0_ref, w1_ref, b1_ref, w2_ref, b2_ref = refs[
        len(refs) - 10:len(refs) - 3]

    @pl.when(i == 0)
    def _():
        gs_acc[...] = jnp.zeros_like(gs_acc)
        cnt_acc[...] = jnp.zeros_like(cnt_acc)

    bid = b_ref[...]
    gi = lax.broadcasted_iota(jnp.int32, (1, num_graphs), 1)
    oh = (bid == gi).astype(jnp.float32)
    dn = (((0,), (0,)), ((), ()))
    gs_acc[...] += lax.dot_general(oh, h, dn,
                                   preferred_element_type=jnp.float32)
    ones = jnp.ones((oh.shape[0], 8), jnp.float32)
    cnt_acc[...] += lax.dot_general(oh, ones, dn,
                                    preferred_element_type=jnp.float32)

    @pl.when(i == last)
    def _():
        ge = gs_acc[...] / jnp.maximum(cnt_acc[...][:, 0:1], 1.0)
        z = _leaky(jnp.dot(ge, w0_ref[...],
                           preferred_element_type=jnp.float32) + b0_ref[...])
        z = _leaky(jnp.dot(z, w1_ref[...],
                           preferred_element_type=jnp.float32) + b1_ref[...])
        o_ref[...] = (
            jnp.dot(z, w2_ref[...], preferred_element_type=jnp.float32)
            + b2_ref[...]
        )


def _upd_head(sparts, cc, hu, res, wu, g, b, batch2d, mp, num_graphs, bm):
    n, h = hu.shape
    n_cols = len(sparts) // 2
    grid = n // bm
    w2p = jnp.zeros((64, 128), jnp.float32).at[:, 0:1].set(mp["W2"])
    b2p = jnp.zeros((128,), jnp.float32).at[0].set(mp["b2"][0])
    body = functools.partial(
        _upd_head_body, aggr_w=h, n_cols=n_cols, num_graphs=num_graphs,
        last=grid - 1)
    blk = lambda w_: pl.BlockSpec((bm, w_), lambda i: (i, 0))
    rep = lambda r_, w_: pl.BlockSpec((r_, w_), lambda i: (0, 0))
    in_specs = [blk(s.shape[1]) for s in sparts]
    in_specs += [blk(16), blk(16), blk(h), blk(h), rep(h, h), rep(1, h),
                 rep(1, h), blk(1), rep(h, 128), rep(1, 128), rep(128, 64),
                 rep(1, 64), rep(64, 128), rep(1, 128)]
    out = pl.pallas_call(
        body,
        grid=(grid,),
        in_specs=in_specs,
        out_specs=pl.BlockSpec((num_graphs, 128), lambda i: (0, 0)),
        out_shape=jax.ShapeDtypeStruct((num_graphs, 128), jnp.float32),
        scratch_shapes=[
            pltpu.VMEM((num_graphs, h), jnp.float32),
            pltpu.VMEM((num_graphs, 8), jnp.float32),
        ],
    )(*sparts, *cc, hu, res, wu, g.reshape(1, h), b.reshape(1, h), batch2d,
      mp["W0"], mp["b0"].reshape(1, 128), mp["W1"], mp["b1"].reshape(1, 64),
      w2p, b2p.reshape(1, 128))
    return out[:, 0:1]


def _sc_edge_body(tab_ref, ew_ref, src_ref, dst_ref, zer_ref,
                  out0_ref, out1_ref,
                  isrc, idst, rows0, rows1, rows2, shared,
                  ew_sems, g_sems, sc_sems,
                  *, width, comp_w, n_chunks, max_cw, rows_per_tile):
    cid = lax.axis_index("c")
    sid = lax.axis_index("s")
    wid = sid * _NC + cid
    rows_bufs = (rows0, rows1, rows2)

    pltpu.sync_copy(zer_ref,
                    shared.at[pl.ds(sid * rows_per_tile, rows_per_tile)])

    lo = (n_chunks * wid) // _NW
    hi = (n_chunks * (wid + 1)) // _NW
    cw = hi - lo
    plsc.subcore_barrier()

    def ew_start(t, s):
        @pl.when(t < cw)
        def _():
            @pl.when(t >= 3)
            def _():
                pltpu.make_async_copy(rows_bufs[s], shared.at[idst.at[s]],
                                      sc_sems[s]).wait()
            base = (lo + t) * _B
            pltpu.async_copy(src_ref.at[pl.ds(base, _B)], isrc.at[s],
                             ew_sems[s])
            pltpu.async_copy(dst_ref.at[pl.ds(base, _B)], idst.at[s],
                             ew_sems[s])
            pltpu.async_copy(ew_ref.at[pl.ds(base, _B)], rows_bufs[s],
                             ew_sems[s])

    def gather_start(t, s):
        @pl.when(t < cw)
        def _():
            base = (lo + t) * _B
            pltpu.make_async_copy(src_ref.at[pl.ds(base, _B)], isrc.at[s],
                                  ew_sems[s]).wait()
            pltpu.make_async_copy(dst_ref.at[pl.ds(base, _B)], idst.at[s],
                                  ew_sems[s]).wait()
            pltpu.make_async_copy(ew_ref.at[pl.ds(base, _B)], rows_bufs[s],
                                  ew_sems[s]).wait()
            pltpu.async_copy(tab_ref.at[isrc.at[s]], rows_bufs[s],
                             g_sems[s], add=True)

    def compute(t, s):
        @pl.when(jnp.logical_and(t >= 0, t < cw))
        def _():
            pltpu.make_async_copy(tab_ref.at[isrc.at[s]], rows_bufs[s],
                                  g_sems[s]).wait()
            buf = rows_bufs[s]

            def row_body(r, cc):
                for col in range(comp_w // 16):
                    sl = pl.ds(col * 16, 16)
                    t0 = buf[r, sl]
                    buf[r, sl] = jnp.maximum(t0, 0.2 * t0)
                return cc

            lax.fori_loop(0, _B, row_body, 0, unroll=2)
            pltpu.async_copy(buf, shared.at[idst.at[s]], sc_sems[s],
                             add=True)

    ew_start(jnp.int32(0), 0)

    def group_body(u, carry):
        for si in range(3):
            t = 3 * u + si
            gather_start(t, si)
            ew_start(t + 1, (si + 1) % 3)
            compute(t - 1, (si + 2) % 3)
        return carry

    n_groups = (max_cw + 1 + 2) // 3
    lax.fori_loop(0, n_groups, group_body, 0)

    for s in range(3):
        pltpu.make_async_copy(rows_bufs[s], shared.at[idst.at[s]],
                              sc_sems[s]).wait()
    plsc.subcore_barrier()

    row0 = sid * rows_per_tile

    @pl.when(cid == 0)
    def _():
        pltpu.sync_copy(shared.at[pl.ds(row0, rows_per_tile)],
                        out0_ref.at[pl.ds(row0, rows_per_tile)])

    @pl.when(cid == 1)
    def _():
        pltpu.sync_copy(shared.at[pl.ds(row0, rows_per_tile)],
                        out1_ref.at[pl.ds(row0, rows_per_tile)])


def _sc_edge(tab, ew, src, dst, comp_w=None):
    n_nodes, width = tab.shape
    comp_w = width if comp_w is None else comp_w
    n_edges = src.shape[0]
    n_chunks = n_edges // _B
    max_cw = -(-n_chunks // _NW)
    rows_per_tile = -(-n_nodes // (8 * _NS)) * 8
    n_pad = rows_per_tile * _NS
    mesh = plsc.VectorSubcoreMesh(core_axis_name="c", subcore_axis_name="s",
                                  num_cores=_NC, num_subcores=_NS)
    body = functools.partial(
        _sc_edge_body, width=width, comp_w=comp_w, n_chunks=n_chunks,
        max_cw=max_cw, rows_per_tile=rows_per_tile)
    zer = jnp.zeros((rows_per_tile, width), jnp.float32)
    s0, s1 = pl.kernel(
        body,
        out_type=(jax.ShapeDtypeStruct((n_pad, width), jnp.float32),) * 2,
        mesh=mesh,
        scratch_types=[
            pltpu.VMEM((3, _B), jnp.int32),
            pltpu.VMEM((3, _B), jnp.int32),
            pltpu.VMEM((_B, width), jnp.float32),
            pltpu.VMEM((_B, width), jnp.float32),
            pltpu.VMEM((_B, width), jnp.float32),
            pltpu.VMEM_SHARED((n_pad, width), jnp.float32),
            [pltpu.SemaphoreType.DMA] * 3,
            [pltpu.SemaphoreType.DMA] * 3,
            [pltpu.SemaphoreType.DMA] * 3,
        ],
    )(tab, ew, src, dst, zer)
    return s0, s1


def _layer_weights(p, din, hdim, de):
    w_x = p["msg_W"][:din]
    w_e = p["msg_W"][din:]
    if hdim < 128:
        padw = 128 - hdim
        tabw = 128
        tab_w = jnp.concatenate([w_x, jnp.zeros((din, padw))], axis=1)
        tab_b = jnp.zeros((tabw,), jnp.float32).at[hdim].set(1.0)
        ew_w = jnp.concatenate([w_e, jnp.zeros((de, padw))], axis=1)
        ew_b = jnp.concatenate([p["msg_b"], jnp.zeros((padw,), jnp.float32)])
    else:
        tabw = hdim
        tab_w = w_x
        tab_b = jnp.zeros((tabw,), jnp.float32)
        ew_w = w_e
        ew_b = p["msg_b"]
    cat_w = jnp.concatenate([tab_w, p["upd_W"][hdim:], p["res_W"]], axis=1)
    cat_b = jnp.concatenate([tab_b, p["upd_b"], p["res_b"]])
    tab_widths = [128] * (tabw // 128)
    return cat_w, cat_b, ew_w, ew_b, tab_widths


def kernel(x, edge_index, edge_attr, batch, params):
    src = edge_index[0]
    dst = edge_index[1]
    n, dn = x.shape
    de = edge_attr.shape[1]
    hid = (64, 128, 256)
    bm_n = 1000
    p0, p1, p2 = (params[f"layer{i}"] for i in range(3))

    cw0, cb0, eww0, ewb0, tw0 = _layer_weights(p0, dn, hid[0], de)
    cw1, cb1, eww1, ewb1, tw1 = _layer_weights(p1, hid[0], hid[1], de)
    cw2, cb2, eww2, ewb2, tw2 = _layer_weights(p2, hid[1], hid[2], de)

    ew_w = jnp.concatenate([eww0, eww1, eww2], axis=1)
    ew_b = jnp.concatenate([ewb0, ewb1, ewb2])
    ew0, ew1, ew2a, ew2b = _mm_multi(edge_attr, ew_w, ew_b,
                                     [128, 128, 128, 128], 1600)

    tab0, hu0, res0 = _mm_multi(x, cw0, cb0, [128, hid[0], hid[0]], bm_n)
    s0_0, s1_0 = _sc_edge(tab0, ew0, src, dst, comp_w=hid[0] + 16)
    c0 = s0_0[:, hid[0]:hid[0] + 16]
    c1 = s1_0[:, hid[0]:hid[0] + 16]

    tab1, hu1, res1 = _upd_mm(
        [s0_0, s1_0], None, hu0, res0, p0["upd_W"][:hid[0]], p0["ln_g"],
        p0["ln_b"], cw1, cb1, [128, hid[1], hid[1]], hid[0], bm_n)

    s0_1, s1_1 = _sc_edge(tab1, ew1, src, dst)

    tab2a, tab2b, hu2, res2 = _upd_mm(
        [s0_1, s1_1], (c0, c1), hu1, res1, p1["upd_W"][:hid[1]], p1["ln_g"],
        p1["ln_b"], cw2, cb2, [128, 128, hid[2], hid[2]], hid[1], bm_n)

    s0_2a, s1_2a = _sc_edge(tab2a, ew2a, src, dst)
    s0_2b, s1_2b = _sc_edge(tab2b, ew2b, src, dst)

    batch2d = batch.reshape(n, 1)
    return _upd_head(
        [s0_2a, s0_2b, s1_2a, s1_2b], (c0, c1), hu2, res2,
        p2["upd_W"][:hid[2]], p2["ln_g"], p2["ln_b"], batch2d,
        params["mlp"], 64, bm_n)


# speedup vs baseline: 1.1541x; 1.0410x over previous
import functools

import jax
import jax.numpy as jnp
from jax import lax
from jax.experimental import pallas as pl
from jax.experimental.pallas import tpu as pltpu
from jax.experimental.pallas import tpu_sc as plsc

_NC = 2
_NS = 16
_NW = _NC * _NS
_B = 128


def _leaky(v):
    return jnp.where(v >= 0, v, 0.2 * v)


def _mm_multi_body(a_ref, w_ref, b_ref, *o_refs, widths):
    full = (
        jnp.dot(a_ref[...], w_ref[...], preferred_element_type=jnp.float32)
        + b_ref[...]
    )
    off = 0
    for o_ref, w_ in zip(o_refs, widths):
        o_ref[...] = full[:, off:off + w_]
        off += w_


def _mm_multi(a, w, b, widths, bm):
    n, k = a.shape
    m = w.shape[1]
    body = functools.partial(_mm_multi_body, widths=tuple(widths))
    return pl.pallas_call(
        body,
        grid=(n // bm,),
        in_specs=[
            pl.BlockSpec((bm, k), lambda i: (i, 0)),
            pl.BlockSpec((k, m), lambda i: (0, 0)),
            pl.BlockSpec((1, m), lambda i: (0, 0)),
        ],
        out_specs=[pl.BlockSpec((bm, w_), lambda i: (i, 0)) for w_ in widths],
        out_shape=[jax.ShapeDtypeStruct((n, w_), jnp.float32)
                   for w_ in widths],
    )(a, w, b.reshape(1, m))


def _aggr_update(refs, aggr_w, cnt_from_s, n_cols):
    sparts = refs[:2 * n_cols]
    refs = refs[2 * n_cols:]
    msums = [sparts[k][...] + sparts[n_cols + k][...] for k in range(n_cols)]
    if cnt_from_s:
        cnt = jnp.maximum(msums[0][:, aggr_w:aggr_w + 1], 1.0)
        aggr = msums[0][:, :aggr_w]
    else:
        c0_ref, c1_ref = refs[:2]
        refs = refs[2:]
        cnt = jnp.maximum(c0_ref[...][:, 0:1] + c1_ref[...][:, 0:1], 1.0)
        aggr = msums[0] if n_cols == 1 else jnp.concatenate(msums, axis=1)
    aggr = aggr / cnt
    hu_ref, res_ref, wu_ref, g_ref, b_ref = refs[:5]
    u = (
        jnp.dot(aggr, wu_ref[...], preferred_element_type=jnp.float32)
        + hu_ref[...]
    )
    hn = _leaky(u) + res_ref[...]
    mu = jnp.mean(hn, axis=-1, keepdims=True)
    var = jnp.mean((hn - mu) ** 2, axis=-1, keepdims=True)
    h = (hn - mu) * lax.rsqrt(var + 1e-5) * g_ref[...] + b_ref[...]
    return h, refs[5:]


def _upd_mm_body(*refs, aggr_w, cnt_from_s, n_cols, widths):
    n_out = len(widths)
    h, rest = _aggr_update(refs[:len(refs) - n_out], aggr_w, cnt_from_s,
                           n_cols)
    cw_ref, cb_ref = rest
    _mm_multi_body_from(h, cw_ref, cb_ref, refs[len(refs) - n_out:], widths)


def _mm_multi_body_from(a, w_ref, b_ref, o_refs, widths):
    full = (
        jnp.dot(a, w_ref[...], preferred_element_type=jnp.float32)
        + b_ref[...]
    )
    off = 0
    for o_ref, w_ in zip(o_refs, widths):
        o_ref[...] = full[:, off:off + w_]
        off += w_


def _upd_mm(sparts, cc, hu, res, wu, g, b, cat_w, cat_b, widths,
            aggr_w, bm):
    n, h = hu.shape
    n_cols = len(sparts) // 2
    cnt_from_s = cc is None
    outw = cat_w.shape[1]
    body = functools.partial(
        _upd_mm_body, aggr_w=aggr_w, cnt_from_s=cnt_from_s, n_cols=n_cols,
        widths=tuple(widths))
    blk = lambda w_: pl.BlockSpec((bm, w_), lambda i: (i, 0))
    rep = lambda r_, w_: pl.BlockSpec((r_, w_), lambda i: (0, 0))
    in_specs = [blk(s.shape[1]) for s in sparts]
    args = list(sparts)
    if not cnt_from_s:
        in_specs += [blk(16), blk(16)]
        args += list(cc)
    in_specs += [blk(h), blk(h), rep(aggr_w, h), rep(1, h), rep(1, h),
                 rep(h, outw), rep(1, outw)]
    args += [hu, res, wu, g.reshape(1, h), b.reshape(1, h), cat_w,
             cat_b.reshape(1, outw)]
    return pl.pallas_call(
        body,
        grid=(n // bm,),
        in_specs=in_specs,
        out_specs=[pl.BlockSpec((bm, w_), lambda i: (i, 0)) for w_ in widths],
        out_shape=[jax.ShapeDtypeStruct((n, w_), jnp.float32)
                   for w_ in widths],
    )(*args)


def _upd_head_body(*refs, aggr_w, n_cols, num_graphs, last):
    i = pl.program_id(0)
    gs_acc, cnt_acc = refs[-2:]
    o_ref = refs[-3]
    h, rest = _aggr_update(refs[:len(refs) - 10], aggr_w, False, n_cols)
    b_ref, w0_ref, b0_ref, w1_ref, b1_ref, w2_ref, b2_ref = refs[
        len(refs) - 10:len(refs) - 3]

    @pl.when(i == 0)
    def _():
        gs_acc[...] = jnp.zeros_like(gs_acc)
        cnt_acc[...] = jnp.zeros_like(cnt_acc)

    bid = b_ref[...]
    gi = lax.broadcasted_iota(jnp.int32, (1, num_graphs), 1)
    oh = (bid == gi).astype(jnp.float32)
    dn = (((0,), (0,)), ((), ()))
    gs_acc[...] += lax.dot_general(oh, h, dn,
                                   preferred_element_type=jnp.float32)
    ones = jnp.ones((oh.shape[0], 8), jnp.float32)
    cnt_acc[...] += lax.dot_general(oh, ones, dn,
                                    preferred_element_type=jnp.float32)

    @pl.when(i == last)
    def _():
        ge = gs_acc[...] / jnp.maximum(cnt_acc[...][:, 0:1], 1.0)
        z = _leaky(jnp.dot(ge, w0_ref[...],
                           preferred_element_type=jnp.float32) + b0_ref[...])
        z = _leaky(jnp.dot(z, w1_ref[...],
                           preferred_element_type=jnp.float32) + b1_ref[...])
        o_ref[...] = (
            jnp.dot(z, w2_ref[...], preferred_element_type=jnp.float32)
            + b2_ref[...]
        )


def _upd_head(sparts, cc, hu, res, wu, g, b, batch2d, mp, num_graphs, bm):
    n, h = hu.shape
    n_cols = len(sparts) // 2
    grid = n // bm
    w2p = jnp.zeros((64, 128), jnp.float32).at[:, 0:1].set(mp["W2"])
    b2p = jnp.zeros((128,), jnp.float32).at[0].set(mp["b2"][0])
    body = functools.partial(
        _upd_head_body, aggr_w=h, n_cols=n_cols, num_graphs=num_graphs,
        last=grid - 1)
    blk = lambda w_: pl.BlockSpec((bm, w_), lambda i: (i, 0))
    rep = lambda r_, w_: pl.BlockSpec((r_, w_), lambda i: (0, 0))
    in_specs = [blk(s.shape[1]) for s in sparts]
    in_specs += [blk(16), blk(16), blk(h), blk(h), rep(h, h), rep(1, h),
                 rep(1, h), blk(1), rep(h, 128), rep(1, 128), rep(128, 64),
                 rep(1, 64), rep(64, 128), rep(1, 128)]
    out = pl.pallas_call(
        body,
        grid=(grid,),
        in_specs=in_specs,
        out_specs=pl.BlockSpec((num_graphs, 128), lambda i: (0, 0)),
        out_shape=jax.ShapeDtypeStruct((num_graphs, 128), jnp.float32),
        scratch_shapes=[
            pltpu.VMEM((num_graphs, h), jnp.float32),
            pltpu.VMEM((num_graphs, 8), jnp.float32),
        ],
    )(*sparts, *cc, hu, res, wu, g.reshape(1, h), b.reshape(1, h), batch2d,
      mp["W0"], mp["b0"].reshape(1, 128), mp["W1"], mp["b1"].reshape(1, 64),
      w2p, b2p.reshape(1, 128))
    return out[:, 0:1]


def _sc_edge_body(tab_ref, ew_ref, src_ref, dst_ref, zer_ref,
                  out0_ref, out1_ref,
                  isrc, idst, rows0, rows1, rows2, shared,
                  ew_sems, g_sems, sc_sems,
                  *, width, comp_w, n_chunks, max_cw, rows_per_tile):
    cid = lax.axis_index("c")
    sid = lax.axis_index("s")
    wid = sid * _NC + cid
    rows_bufs = (rows0, rows1, rows2)

    pltpu.sync_copy(zer_ref,
                    shared.at[pl.ds(sid * rows_per_tile, rows_per_tile)])

    lo = (n_chunks * wid) // _NW
    hi = (n_chunks * (wid + 1)) // _NW
    cw = hi - lo
    plsc.subcore_barrier()

    def ew_start(t, s):
        @pl.when(t < cw)
        def _():
            @pl.when(t >= 3)
            def _():
                pltpu.make_async_copy(rows_bufs[s], shared.at[idst.at[s]],
                                      sc_sems[s]).wait()
            base = (lo + t) * _B
            pltpu.async_copy(src_ref.at[pl.ds(base, _B)], isrc.at[s],
                             ew_sems[s])
            pltpu.async_copy(dst_ref.at[pl.ds(base, _B)], idst.at[s],
                             ew_sems[s])
            pltpu.async_copy(ew_ref.at[pl.ds(base, _B)], rows_bufs[s],
                             ew_sems[s])

    def gather_start(t, s):
        @pl.when(t < cw)
        def _():
            base = (lo + t) * _B
            pltpu.make_async_copy(src_ref.at[pl.ds(base, _B)], isrc.at[s],
                                  ew_sems[s]).wait()
            pltpu.make_async_copy(dst_ref.at[pl.ds(base, _B)], idst.at[s],
                                  ew_sems[s]).wait()
            pltpu.make_async_copy(ew_ref.at[pl.ds(base, _B)], rows_bufs[s],
                                  ew_sems[s]).wait()
            pltpu.async_copy(tab_ref.at[isrc.at[s]], rows_bufs[s],
                             g_sems[s], add=True)

    def compute(t, s):
        @pl.when(jnp.logical_and(t >= 0, t < cw))
        def _():
            pltpu.make_async_copy(tab_ref.at[isrc.at[s]], rows_bufs[s],
                                  g_sems[s]).wait()
            buf = rows_bufs[s]

            def row_body(r, cc):
                for col in range(comp_w // 16):
                    sl = pl.ds(col * 16, 16)
                    t0 = buf[r, sl]
                    buf[r, sl] = jnp.maximum(t0, 0.2 * t0)
                return cc

            lax.fori_loop(0, _B, row_body, 0, unroll=2)
            pltpu.async_copy(buf, shared.at[idst.at[s]], sc_sems[s],
                             add=True)

    ew_start(jnp.int32(0), 0)

    def group_body(u, carry):
        for si in range(3):
            t = 3 * u + si
            gather_start(t, si)
            ew_start(t + 1, (si + 1) % 3)
            compute(t - 1, (si + 2) % 3)
        return carry

    n_groups = (max_cw + 1 + 2) // 3
    lax.fori_loop(0, n_groups, group_body, 0)

    for s in range(3):
        pltpu.make_async_copy(rows_bufs[s], shared.at[idst.at[s]],
                              sc_sems[s]).wait()
    plsc.subcore_barrier()

    row0 = sid * rows_per_tile

    @pl.when(cid == 0)
    def _():
        pltpu.sync_copy(shared.at[pl.ds(row0, rows_per_tile)],
                        out0_ref.at[pl.ds(row0, rows_per_tile)])

    @pl.when(cid == 1)
    def _():
        pltpu.sync_copy(shared.at[pl.ds(row0, rows_per_tile)],
                        out1_ref.at[pl.ds(row0, rows_per_tile)])


def _sc_edge(tab, ew, src, dst, comp_w=None):
    n_nodes, width = tab.shape
    comp_w = width if comp_w is None else comp_w
    n_edges = src.shape[0]
    n_chunks = n_edges // _B
    max_cw = -(-n_chunks // _NW)
    rows_per_tile = -(-n_nodes // (8 * _NS)) * 8
    n_pad = rows_per_tile * _NS
    mesh = plsc.VectorSubcoreMesh(core_axis_name="c", subcore_axis_name="s",
                                  num_cores=_NC, num_subcores=_NS)
    body = functools.partial(
        _sc_edge_body, width=width, comp_w=comp_w, n_chunks=n_chunks,
        max_cw=max_cw, rows_per_tile=rows_per_tile)
    zer = jnp.zeros((rows_per_tile, width), jnp.float32)
    s0, s1 = pl.kernel(
        body,
        out_type=(jax.ShapeDtypeStruct((n_pad, width), jnp.float32),) * 2,
        mesh=mesh,
        scratch_types=[
            pltpu.VMEM((3, _B), jnp.int32),
            pltpu.VMEM((3, _B), jnp.int32),
            pltpu.VMEM((_B, width), jnp.float32),
            pltpu.VMEM((_B, width), jnp.float32),
            pltpu.VMEM((_B, width), jnp.float32),
            pltpu.VMEM_SHARED((n_pad, width), jnp.float32),
            [pltpu.SemaphoreType.DMA] * 3,
            [pltpu.SemaphoreType.DMA] * 3,
            [pltpu.SemaphoreType.DMA] * 3,
        ],
    )(tab, ew, src, dst, zer)
    return s0, s1


def _layer_weights(p, din, hdim, de):
    w_x = p["msg_W"][:din]
    w_e = p["msg_W"][din:]
    if hdim < 128:
        padw = 128 - hdim
        tabw = 128
        tab_w = jnp.concatenate([w_x, jnp.zeros((din, padw))], axis=1)
        tab_b = jnp.zeros((tabw,), jnp.float32).at[hdim].set(1.0)
        ew_w = jnp.concatenate([w_e, jnp.zeros((de, padw))], axis=1)
        ew_b = jnp.concatenate([p["msg_b"], jnp.zeros((padw,), jnp.float32)])
    else:
        tabw = hdim
        tab_w = w_x
        tab_b = jnp.zeros((tabw,), jnp.float32)
        ew_w = w_e
        ew_b = p["msg_b"]
    cat_w = jnp.concatenate([tab_w, p["upd_W"][hdim:], p["res_W"]], axis=1)
    cat_b = jnp.concatenate([tab_b, p["upd_b"], p["res_b"]])
    tab_widths = [128] * (tabw // 128)
    return cat_w, cat_b, ew_w, ew_b, tab_widths


def kernel(x, edge_index, edge_attr, batch, params):
    src = edge_index[0]
    dst = edge_index[1]
    n, dn = x.shape
    de = edge_attr.shape[1]
    hid = (64, 128, 256)
    bm_n = 2000
    p0, p1, p2 = (params[f"layer{i}"] for i in range(3))

    cw0, cb0, eww0, ewb0, tw0 = _layer_weights(p0, dn, hid[0], de)
    cw1, cb1, eww1, ewb1, tw1 = _layer_weights(p1, hid[0], hid[1], de)
    cw2, cb2, eww2, ewb2, tw2 = _layer_weights(p2, hid[1], hid[2], de)

    ew_w = jnp.concatenate([eww0, eww1, eww2], axis=1)
    ew_b = jnp.concatenate([ewb0, ewb1, ewb2])
    ew0, ew1, ew2a, ew2b = _mm_multi(edge_attr, ew_w, ew_b,
                                     [128, 128, 128, 128], 3200)

    tab0, hu0, res0 = _mm_multi(x, cw0, cb0, [128, hid[0], hid[0]], bm_n)
    s0_0, s1_0 = _sc_edge(tab0, ew0, src, dst, comp_w=hid[0] + 16)
    c0 = s0_0[:, hid[0]:hid[0] + 16]
    c1 = s1_0[:, hid[0]:hid[0] + 16]

    tab1, hu1, res1 = _upd_mm(
        [s0_0, s1_0], None, hu0, res0, p0["upd_W"][:hid[0]], p0["ln_g"],
        p0["ln_b"], cw1, cb1, [128, hid[1], hid[1]], hid[0], bm_n)

    s0_1, s1_1 = _sc_edge(tab1, ew1, src, dst)

    tab2a, tab2b, hu2, res2 = _upd_mm(
        [s0_1, s1_1], (c0, c1), hu1, res1, p1["upd_W"][:hid[1]], p1["ln_g"],
        p1["ln_b"], cw2, cb2, [128, 128, hid[2], hid[2]], hid[1], bm_n)

    s0_2a, s1_2a = _sc_edge(tab2a, ew2a, src, dst)
    s0_2b, s1_2b = _sc_edge(tab2b, ew2b, src, dst)

    batch2d = batch.reshape(n, 1)
    return _upd_head(
        [s0_2a, s0_2b, s1_2a, s1_2b], (c0, c1), hu2, res2,
        p2["upd_W"][:hid[2]], p2["ln_g"], p2["ln_b"], batch2d,
        params["mlp"], 64, bm_n)


# speedup vs baseline: 1.1619x; 1.0067x over previous
import functools

import jax
import jax.numpy as jnp
from jax import lax
from jax.experimental import pallas as pl
from jax.experimental.pallas import tpu as pltpu
from jax.experimental.pallas import tpu_sc as plsc

_NC = 2
_NS = 16
_NW = _NC * _NS
_B = 128


def _leaky(v):
    return jnp.where(v >= 0, v, 0.2 * v)


def _mm_multi_body(a_ref, w_ref, b_ref, *o_refs, widths):
    full = (
        jnp.dot(a_ref[...], w_ref[...], preferred_element_type=jnp.float32)
        + b_ref[...]
    )
    off = 0
    for o_ref, w_ in zip(o_refs, widths):
        o_ref[...] = full[:, off:off + w_]
        off += w_


def _mm_multi(a, w, b, widths, bm):
    n, k = a.shape
    m = w.shape[1]
    body = functools.partial(_mm_multi_body, widths=tuple(widths))
    return pl.pallas_call(
        body,
        grid=(n // bm,),
        in_specs=[
            pl.BlockSpec((bm, k), lambda i: (i, 0)),
            pl.BlockSpec((k, m), lambda i: (0, 0)),
            pl.BlockSpec((1, m), lambda i: (0, 0)),
        ],
        out_specs=[pl.BlockSpec((bm, w_), lambda i: (i, 0)) for w_ in widths],
        out_shape=[jax.ShapeDtypeStruct((n, w_), jnp.float32)
                   for w_ in widths],
    )(a, w, b.reshape(1, m))


def _aggr_update(refs, aggr_w, cnt_from_s, n_cols):
    sparts = refs[:2 * n_cols]
    refs = refs[2 * n_cols:]
    msums = [sparts[k][...] + sparts[n_cols + k][...] for k in range(n_cols)]
    if cnt_from_s:
        cnt = jnp.maximum(msums[0][:, aggr_w:aggr_w + 1], 1.0)
        aggr = msums[0][:, :aggr_w]
    else:
        c0_ref, c1_ref = refs[:2]
        refs = refs[2:]
        cnt = jnp.maximum(c0_ref[...][:, 0:1] + c1_ref[...][:, 0:1], 1.0)
        aggr = msums[0] if n_cols == 1 else jnp.concatenate(msums, axis=1)
    aggr = aggr / cnt
    hu_ref, res_ref, wu_ref, g_ref, b_ref = refs[:5]
    u = (
        jnp.dot(aggr, wu_ref[...], preferred_element_type=jnp.float32)
        + hu_ref[...]
    )
    hn = _leaky(u) + res_ref[...]
    mu = jnp.mean(hn, axis=-1, keepdims=True)
    var = jnp.mean((hn - mu) ** 2, axis=-1, keepdims=True)
    h = (hn - mu) * lax.rsqrt(var + 1e-5) * g_ref[...] + b_ref[...]
    return h, refs[5:]


def _upd_mm_body(*refs, aggr_w, cnt_from_s, n_cols, widths):
    n_out = len(widths)
    h, rest = _aggr_update(refs[:len(refs) - n_out], aggr_w, cnt_from_s,
                           n_cols)
    cw_ref, cb_ref = rest
    _mm_multi_body_from(h, cw_ref, cb_ref, refs[len(refs) - n_out:], widths)


def _mm_multi_body_from(a, w_ref, b_ref, o_refs, widths):
    full = (
        jnp.dot(a, w_ref[...], preferred_element_type=jnp.float32)
        + b_ref[...]
    )
    off = 0
    for o_ref, w_ in zip(o_refs, widths):
        o_ref[...] = full[:, off:off + w_]
        off += w_


def _upd_mm(sparts, cc, hu, res, wu, g, b, cat_w, cat_b, widths,
            aggr_w, bm):
    n, h = hu.shape
    n_cols = len(sparts) // 2
    cnt_from_s = cc is None
    outw = cat_w.shape[1]
    body = functools.partial(
        _upd_mm_body, aggr_w=aggr_w, cnt_from_s=cnt_from_s, n_cols=n_cols,
        widths=tuple(widths))
    blk = lambda w_: pl.BlockSpec((bm, w_), lambda i: (i, 0))
    rep = lambda r_, w_: pl.BlockSpec((r_, w_), lambda i: (0, 0))
    in_specs = [blk(s.shape[1]) for s in sparts]
    args = list(sparts)
    if not cnt_from_s:
        in_specs += [blk(16), blk(16)]
        args += list(cc)
    in_specs += [blk(h), blk(h), rep(aggr_w, h), rep(1, h), rep(1, h),
                 rep(h, outw), rep(1, outw)]
    args += [hu, res, wu, g.reshape(1, h), b.reshape(1, h), cat_w,
             cat_b.reshape(1, outw)]
    return pl.pallas_call(
        body,
        grid=(n // bm,),
        in_specs=in_specs,
        out_specs=[pl.BlockSpec((bm, w_), lambda i: (i, 0)) for w_ in widths],
        out_shape=[jax.ShapeDtypeStruct((n, w_), jnp.float32)
                   for w_ in widths],
    )(*args)


def _upd_head_body(*refs, aggr_w, n_cols, num_graphs, last):
    i = pl.program_id(0)
    gs_acc, cnt_acc = refs[-2:]
    o_ref = refs[-3]
    h, rest = _aggr_update(refs[:len(refs) - 10], aggr_w, False, n_cols)
    b_ref, w0_ref, b0_ref, w1_ref, b1_ref, w2_ref, b2_ref = refs[
        len(refs) - 10:len(refs) - 3]

    @pl.when(i == 0)
    def _():
        gs_acc[...] = jnp.zeros_like(gs_acc)
        cnt_acc[...] = jnp.zeros_like(cnt_acc)

    bid = b_ref[...]
    gi = lax.broadcasted_iota(jnp.int32, (1, num_graphs), 1)
    oh = (bid == gi).astype(jnp.float32)
    dn = (((0,), (0,)), ((), ()))
    gs_acc[...] += lax.dot_general(oh, h, dn,
                                   preferred_element_type=jnp.float32)
    ones = jnp.ones((oh.shape[0], 8), jnp.float32)
    cnt_acc[...] += lax.dot_general(oh, ones, dn,
                                    preferred_element_type=jnp.float32)

    @pl.when(i == last)
    def _():
        ge = gs_acc[...] / jnp.maximum(cnt_acc[...][:, 0:1], 1.0)
        z = _leaky(jnp.dot(ge, w0_ref[...],
                           preferred_element_type=jnp.float32) + b0_ref[...])
        z = _leaky(jnp.dot(z, w1_ref[...],
                           preferred_element_type=jnp.float32) + b1_ref[...])
        o_ref[...] = (
            jnp.dot(z, w2_ref[...], preferred_element_type=jnp.float32)
            + b2_ref[...]
        )


def _upd_head(sparts, cc, hu, res, wu, g, b, batch2d, mp, num_graphs, bm):
    n, h = hu.shape
    n_cols = len(sparts) // 2
    grid = n // bm
    w2p = jnp.zeros((64, 128), jnp.float32).at[:, 0:1].set(mp["W2"])
    b2p = jnp.zeros((128,), jnp.float32).at[0].set(mp["b2"][0])
    body = functools.partial(
        _upd_head_body, aggr_w=h, n_cols=n_cols, num_graphs=num_graphs,
        last=grid - 1)
    blk = lambda w_: pl.BlockSpec((bm, w_), lambda i: (i, 0))
    rep = lambda r_, w_: pl.BlockSpec((r_, w_), lambda i: (0, 0))
    in_specs = [blk(s.shape[1]) for s in sparts]
    in_specs += [blk(16), blk(16), blk(h), blk(h), rep(h, h), rep(1, h),
                 rep(1, h), blk(1), rep(h, 128), rep(1, 128), rep(128, 64),
                 rep(1, 64), rep(64, 128), rep(1, 128)]
    out = pl.pallas_call(
        body,
        grid=(grid,),
        in_specs=in_specs,
        out_specs=pl.BlockSpec((num_graphs, 128), lambda i: (0, 0)),
        out_shape=jax.ShapeDtypeStruct((num_graphs, 128), jnp.float32),
        scratch_shapes=[
            pltpu.VMEM((num_graphs, h), jnp.float32),
            pltpu.VMEM((num_graphs, 8), jnp.float32),
        ],
    )(*sparts, *cc, hu, res, wu, g.reshape(1, h), b.reshape(1, h), batch2d,
      mp["W0"], mp["b0"].reshape(1, 128), mp["W1"], mp["b1"].reshape(1, 64),
      w2p, b2p.reshape(1, 128))
    return out[:, 0:1]


def _sc_edge_body(tab_ref, ew_ref, src_ref, dst_ref, zer_ref,
                  out0_ref, out1_ref,
                  isrc, idst, rows0, rows1, rows2, shared,
                  ew_sems, g_sems, sc_sems,
                  *, width, comp_w, n_chunks, max_cw, rows_per_tile):
    cid = lax.axis_index("c")
    sid = lax.axis_index("s")
    wid = sid * _NC + cid
    rows_bufs = (rows0, rows1, rows2)

    pltpu.sync_copy(zer_ref,
                    shared.at[pl.ds(sid * rows_per_tile, rows_per_tile)])

    lo = (n_chunks * wid) // _NW
    hi = (n_chunks * (wid + 1)) // _NW
    cw = hi - lo
    plsc.subcore_barrier()

    def ew_start(t, s):
        @pl.when(t < cw)
        def _():
            @pl.when(t >= 3)
            def _():
                pltpu.make_async_copy(rows_bufs[s], shared.at[idst.at[s]],
                                      sc_sems[s]).wait()
            base = (lo + t) * _B
            pltpu.async_copy(src_ref.at[pl.ds(base, _B)], isrc.at[s],
                             ew_sems[s])
            pltpu.async_copy(dst_ref.at[pl.ds(base, _B)], idst.at[s],
                             ew_sems[s])
            pltpu.async_copy(ew_ref.at[pl.ds(base, _B)], rows_bufs[s],
                             ew_sems[s])

    def gather_start(t, s):
        @pl.when(t < cw)
        def _():
            base = (lo + t) * _B
            pltpu.make_async_copy(src_ref.at[pl.ds(base, _B)], isrc.at[s],
                                  ew_sems[s]).wait()
            pltpu.make_async_copy(dst_ref.at[pl.ds(base, _B)], idst.at[s],
                                  ew_sems[s]).wait()
            pltpu.make_async_copy(ew_ref.at[pl.ds(base, _B)], rows_bufs[s],
                                  ew_sems[s]).wait()
            pltpu.async_copy(tab_ref.at[isrc.at[s]], rows_bufs[s],
                             g_sems[s], add=True)

    def compute(t, s):
        @pl.when(jnp.logical_and(t >= 0, t < cw))
        def _():
            pltpu.make_async_copy(tab_ref.at[isrc.at[s]], rows_bufs[s],
                                  g_sems[s]).wait()
            buf = rows_bufs[s]

            def row_body(r, cc):
                for col in range(comp_w // 16):
                    sl = pl.ds(col * 16, 16)
                    t0 = buf[r, sl]
                    buf[r, sl] = jnp.maximum(t0, 0.2 * t0)
                return cc

            lax.fori_loop(0, _B, row_body, 0, unroll=2)
            pltpu.async_copy(buf, shared.at[idst.at[s]], sc_sems[s],
                             add=True)

    ew_start(jnp.int32(0), 0)

    def group_body(u, carry):
        for si in range(3):
            t = 3 * u + si
            gather_start(t, si)
            ew_start(t + 1, (si + 1) % 3)
            compute(t - 1, (si + 2) % 3)
        return carry

    n_groups = (max_cw + 1 + 2) // 3
    lax.fori_loop(0, n_groups, group_body, 0)

    for s in range(3):
        pltpu.make_async_copy(rows_bufs[s], shared.at[idst.at[s]],
                              sc_sems[s]).wait()
    plsc.subcore_barrier()

    row0 = sid * rows_per_tile

    @pl.when(cid == 0)
    def _():
        pltpu.sync_copy(shared.at[pl.ds(row0, rows_per_tile)],
                        out0_ref.at[pl.ds(row0, rows_per_tile)])

    @pl.when(cid == 1)
    def _():
        pltpu.sync_copy(shared.at[pl.ds(row0, rows_per_tile)],
                        out1_ref.at[pl.ds(row0, rows_per_tile)])


def _sc_edge(tab, ew, src, dst, comp_w=None):
    n_nodes, width = tab.shape
    comp_w = width if comp_w is None else comp_w
    n_edges = src.shape[0]
    n_chunks = n_edges // _B
    max_cw = -(-n_chunks // _NW)
    rows_per_tile = -(-n_nodes // (8 * _NS)) * 8
    n_pad = rows_per_tile * _NS
    mesh = plsc.VectorSubcoreMesh(core_axis_name="c", subcore_axis_name="s",
                                  num_cores=_NC, num_subcores=_NS)
    body = functools.partial(
        _sc_edge_body, width=width, comp_w=comp_w, n_chunks=n_chunks,
        max_cw=max_cw, rows_per_tile=rows_per_tile)
    zer = jnp.zeros((rows_per_tile, width), jnp.float32)
    s0, s1 = pl.kernel(
        body,
        out_type=(jax.ShapeDtypeStruct((n_pad, width), jnp.float32),) * 2,
        mesh=mesh,
        scratch_types=[
            pltpu.VMEM((3, _B), jnp.int32),
            pltpu.VMEM((3, _B), jnp.int32),
            pltpu.VMEM((_B, width), jnp.float32),
            pltpu.VMEM((_B, width), jnp.float32),
            pltpu.VMEM((_B, width), jnp.float32),
            pltpu.VMEM_SHARED((n_pad, width), jnp.float32),
            [pltpu.SemaphoreType.DMA] * 3,
            [pltpu.SemaphoreType.DMA] * 3,
            [pltpu.SemaphoreType.DMA] * 3,
        ],
    )(tab, ew, src, dst, zer)
    return s0, s1


def _layer_weights(p, din, hdim, de):
    w_x = p["msg_W"][:din]
    w_e = p["msg_W"][din:]
    if hdim < 128:
        padw = 128 - hdim
        tabw = 128
        tab_w = jnp.concatenate([w_x, jnp.zeros((din, padw))], axis=1)
        tab_b = jnp.zeros((tabw,), jnp.float32).at[hdim].set(1.0)
        ew_w = jnp.concatenate([w_e, jnp.zeros((de, padw))], axis=1)
        ew_b = jnp.concatenate([p["msg_b"], jnp.zeros((padw,), jnp.float32)])
    else:
        tabw = hdim
        tab_w = w_x
        tab_b = jnp.zeros((tabw,), jnp.float32)
        ew_w = w_e
        ew_b = p["msg_b"]
    cat_w = jnp.concatenate([tab_w, p["upd_W"][hdim:], p["res_W"]], axis=1)
    cat_b = jnp.concatenate([tab_b, p["upd_b"], p["res_b"]])
    tab_widths = [128] * (tabw // 128)
    return cat_w, cat_b, ew_w, ew_b, tab_widths


def kernel(x, edge_index, edge_attr, batch, params):
    src = edge_index[0]
    dst = edge_index[1]
    n, dn = x.shape
    de = edge_attr.shape[1]
    hid = (64, 128, 256)
    bm_n = 2000
    p0, p1, p2 = (params[f"layer{i}"] for i in range(3))

    cw0, cb0, eww0, ewb0, tw0 = _layer_weights(p0, dn, hid[0], de)
    cw1, cb1, eww1, ewb1, tw1 = _layer_weights(p1, hid[0], hid[1], de)
    cw2, cb2, eww2, ewb2, tw2 = _layer_weights(p2, hid[1], hid[2], de)

    ew_w = jnp.concatenate([eww0, eww1, eww2], axis=1)
    ew_b = jnp.concatenate([ewb0, ewb1, ewb2])
    ew0, ew1, ew2a, ew2b = _mm_multi(edge_attr, ew_w, ew_b,
                                     [128, 128, 128, 128], 6400)

    tab0, hu0, res0 = _mm_multi(x, cw0, cb0, [128, hid[0], hid[0]], bm_n)
    s0_0, s1_0 = _sc_edge(tab0, ew0, src, dst, comp_w=hid[0] + 16)
    c0 = s0_0[:, hid[0]:hid[0] + 16]
    c1 = s1_0[:, hid[0]:hid[0] + 16]

    tab1, hu1, res1 = _upd_mm(
        [s0_0, s1_0], None, hu0, res0, p0["upd_W"][:hid[0]], p0["ln_g"],
        p0["ln_b"], cw1, cb1, [128, hid[1], hid[1]], hid[0], bm_n)

    s0_1, s1_1 = _sc_edge(tab1, ew1, src, dst)

    tab2a, tab2b, hu2, res2 = _upd_mm(
        [s0_1, s1_1], (c0, c1), hu1, res1, p1["upd_W"][:hid[1]], p1["ln_g"],
        p1["ln_b"], cw2, cb2, [128, 128, hid[2], hid[2]], hid[1], bm_n)

    s0_2a, s1_2a = _sc_edge(tab2a, ew2a, src, dst)
    s0_2b, s1_2b = _sc_edge(tab2b, ew2b, src, dst)

    batch2d = batch.reshape(n, 1)
    return _upd_head(
        [s0_2a, s0_2b, s1_2a, s1_2b], (c0, c1), hu2, res2,
        p2["upd_W"][:hid[2]], p2["ln_g"], p2["ln_b"], batch2d,
        params["mlp"], 64, bm_n)


# speedup vs baseline: 1.1672x; 1.0046x over previous
import functools

import jax
import jax.numpy as jnp
from jax import lax
from jax.experimental import pallas as pl
from jax.experimental.pallas import tpu as pltpu
from jax.experimental.pallas import tpu_sc as plsc

_NC = 2
_NS = 16
_NW = _NC * _NS
_B = 128


def _leaky(v):
    return jnp.where(v >= 0, v, 0.2 * v)


def _mm_multi_body(a_ref, w_ref, b_ref, *o_refs, widths):
    full = (
        jnp.dot(a_ref[...], w_ref[...], preferred_element_type=jnp.float32)
        + b_ref[...]
    )
    off = 0
    for o_ref, w_ in zip(o_refs, widths):
        o_ref[...] = full[:, off:off + w_]
        off += w_


def _mm_multi(a, w, b, widths, bm):
    n, k = a.shape
    m = w.shape[1]
    body = functools.partial(_mm_multi_body, widths=tuple(widths))
    return pl.pallas_call(
        body,
        grid=(n // bm,),
        in_specs=[
            pl.BlockSpec((bm, k), lambda i: (i, 0)),
            pl.BlockSpec((k, m), lambda i: (0, 0)),
            pl.BlockSpec((1, m), lambda i: (0, 0)),
        ],
        out_specs=[pl.BlockSpec((bm, w_), lambda i: (i, 0)) for w_ in widths],
        out_shape=[jax.ShapeDtypeStruct((n, w_), jnp.float32)
                   for w_ in widths],
    )(a, w, b.reshape(1, m))


def _aggr_update(refs, aggr_w, cnt_from_s, n_cols):
    sparts = refs[:2 * n_cols]
    refs = refs[2 * n_cols:]
    msums = [sparts[k][...] + sparts[n_cols + k][...] for k in range(n_cols)]
    if cnt_from_s:
        cnt = jnp.maximum(msums[0][:, aggr_w:aggr_w + 1], 1.0)
        aggr = msums[0][:, :aggr_w]
    else:
        c0_ref, c1_ref = refs[:2]
        refs = refs[2:]
        cnt = jnp.maximum(c0_ref[...][:, 0:1] + c1_ref[...][:, 0:1], 1.0)
        aggr = msums[0] if n_cols == 1 else jnp.concatenate(msums, axis=1)
    aggr = aggr / cnt
    hu_ref, res_ref, wu_ref, g_ref, b_ref = refs[:5]
    u = (
        jnp.dot(aggr, wu_ref[...], preferred_element_type=jnp.float32)
        + hu_ref[...]
    )
    hn = _leaky(u) + res_ref[...]
    mu = jnp.mean(hn, axis=-1, keepdims=True)
    var = jnp.mean((hn - mu) ** 2, axis=-1, keepdims=True)
    h = (hn - mu) * lax.rsqrt(var + 1e-5) * g_ref[...] + b_ref[...]
    return h, refs[5:]


def _upd_mm_body(*refs, aggr_w, cnt_from_s, n_cols, widths):
    n_out = len(widths)
    h, rest = _aggr_update(refs[:len(refs) - n_out], aggr_w, cnt_from_s,
                           n_cols)
    cw_ref, cb_ref = rest
    _mm_multi_body_from(h, cw_ref, cb_ref, refs[len(refs) - n_out:], widths)


def _mm_multi_body_from(a, w_ref, b_ref, o_refs, widths):
    full = (
        jnp.dot(a, w_ref[...], preferred_element_type=jnp.float32)
        + b_ref[...]
    )
    off = 0
    for o_ref, w_ in zip(o_refs, widths):
        o_ref[...] = full[:, off:off + w_]
        off += w_


def _upd_mm(sparts, cc, hu, res, wu, g, b, cat_w, cat_b, widths,
            aggr_w, bm):
    n, h = hu.shape
    n_cols = len(sparts) // 2
    cnt_from_s = cc is None
    outw = cat_w.shape[1]
    body = functools.partial(
        _upd_mm_body, aggr_w=aggr_w, cnt_from_s=cnt_from_s, n_cols=n_cols,
        widths=tuple(widths))
    blk = lambda w_: pl.BlockSpec((bm, w_), lambda i: (i, 0))
    rep = lambda r_, w_: pl.BlockSpec((r_, w_), lambda i: (0, 0))
    in_specs = [blk(s.shape[1]) for s in sparts]
    args = list(sparts)
    if not cnt_from_s:
        in_specs += [blk(16), blk(16)]
        args += list(cc)
    in_specs += [blk(h), blk(h), rep(aggr_w, h), rep(1, h), rep(1, h),
                 rep(h, outw), rep(1, outw)]
    args += [hu, res, wu, g.reshape(1, h), b.reshape(1, h), cat_w,
             cat_b.reshape(1, outw)]
    return pl.pallas_call(
        body,
        grid=(n // bm,),
        in_specs=in_specs,
        out_specs=[pl.BlockSpec((bm, w_), lambda i: (i, 0)) for w_ in widths],
        out_shape=[jax.ShapeDtypeStruct((n, w_), jnp.float32)
                   for w_ in widths],
    )(*args)


def _upd_head_body(*refs, aggr_w, n_cols, num_graphs, last):
    i = pl.program_id(0)
    gs_acc, cnt_acc = refs[-2:]
    o_ref = refs[-3]
    h, rest = _aggr_update(refs[:len(refs) - 10], aggr_w, False, n_cols)
    b_ref, w0_ref, b0_ref, w1_ref, b1_ref, w2_ref, b2_ref = refs[
        len(refs) - 10:len(refs) - 3]

    @pl.when(i == 0)
    def _():
        gs_acc[...] = jnp.zeros_like(gs_acc)
        cnt_acc[...] = jnp.zeros_like(cnt_acc)

    bid = b_ref[...]
    gi = lax.broadcasted_iota(jnp.int32, (1, num_graphs), 1)
    oh = (bid == gi).astype(jnp.float32)
    dn = (((0,), (0,)), ((), ()))
    gs_acc[...] += lax.dot_general(oh, h, dn,
                                   preferred_element_type=jnp.float32)
    ones = jnp.ones((oh.shape[0], 8), jnp.float32)
    cnt_acc[...] += lax.dot_general(oh, ones, dn,
                                    preferred_element_type=jnp.float32)

    @pl.when(i == last)
    def _():
        ge = gs_acc[...] / jnp.maximum(cnt_acc[...][:, 0:1], 1.0)
        z = _leaky(jnp.dot(ge, w0_ref[...],
                           preferred_element_type=jnp.float32) + b0_ref[...])
        z = _leaky(jnp.dot(z, w1_ref[...],
                           preferred_element_type=jnp.float32) + b1_ref[...])
        o_ref[...] = (
            jnp.dot(z, w2_ref[...], preferred_element_type=jnp.float32)
            + b2_ref[...]
        )


def _upd_head(sparts, cc, hu, res, wu, g, b, batch2d, mp, num_graphs, bm):
    n, h = hu.shape
    n_cols = len(sparts) // 2
    grid = n // bm
    w2p = jnp.zeros((64, 128), jnp.float32).at[:, 0:1].set(mp["W2"])
    b2p = jnp.zeros((128,), jnp.float32).at[0].set(mp["b2"][0])
    body = functools.partial(
        _upd_head_body, aggr_w=h, n_cols=n_cols, num_graphs=num_graphs,
        last=grid - 1)
    blk = lambda w_: pl.BlockSpec((bm, w_), lambda i: (i, 0))
    rep = lambda r_, w_: pl.BlockSpec((r_, w_), lambda i: (0, 0))
    in_specs = [blk(s.shape[1]) for s in sparts]
    in_specs += [blk(16), blk(16), blk(h), blk(h), rep(h, h), rep(1, h),
                 rep(1, h), blk(1), rep(h, 128), rep(1, 128), rep(128, 64),
                 rep(1, 64), rep(64, 128), rep(1, 128)]
    out = pl.pallas_call(
        body,
        grid=(grid,),
        in_specs=in_specs,
        out_specs=pl.BlockSpec((num_graphs, 128), lambda i: (0, 0)),
        out_shape=jax.ShapeDtypeStruct((num_graphs, 128), jnp.float32),
        scratch_shapes=[
            pltpu.VMEM((num_graphs, h), jnp.float32),
            pltpu.VMEM((num_graphs, 8), jnp.float32),
        ],
    )(*sparts, *cc, hu, res, wu, g.reshape(1, h), b.reshape(1, h), batch2d,
      mp["W0"], mp["b0"].reshape(1, 128), mp["W1"], mp["b1"].reshape(1, 64),
      w2p, b2p.reshape(1, 128))
    return out[:, 0:1]


def _sc_one_pass(tab_ref, ew_ref, src_ref, dst_ref, zer_ref,
                 out0_ref, out1_ref,
                 isrc, idst, rows_bufs, shared,
                 ew_sems, g_sems, sc_sems,
                 cid, sid, lo, cw,
                 *, comp_w, max_cw, rows_per_tile):
    pltpu.sync_copy(zer_ref,
                    shared.at[pl.ds(sid * rows_per_tile, rows_per_tile)])
    plsc.subcore_barrier()

    def ew_start(t, s):
        @pl.when(t < cw)
        def _():
            @pl.when(t >= 3)
            def _():
                pltpu.make_async_copy(rows_bufs[s], shared.at[idst.at[s]],
                                      sc_sems[s]).wait()
            base = (lo + t) * _B
            pltpu.async_copy(src_ref.at[pl.ds(base, _B)], isrc.at[s],
                             ew_sems[s])
            pltpu.async_copy(dst_ref.at[pl.ds(base, _B)], idst.at[s],
                             ew_sems[s])
            pltpu.async_copy(ew_ref.at[pl.ds(base, _B)], rows_bufs[s],
                             ew_sems[s])

    def gather_start(t, s):
        @pl.when(t < cw)
        def _():
            base = (lo + t) * _B
            pltpu.make_async_copy(src_ref.at[pl.ds(base, _B)], isrc.at[s],
                                  ew_sems[s]).wait()
            pltpu.make_async_copy(dst_ref.at[pl.ds(base, _B)], idst.at[s],
                                  ew_sems[s]).wait()
            pltpu.make_async_copy(ew_ref.at[pl.ds(base, _B)], rows_bufs[s],
                                  ew_sems[s]).wait()
            pltpu.async_copy(tab_ref.at[isrc.at[s]], rows_bufs[s],
                             g_sems[s], add=True)

    def compute(t, s):
        @pl.when(jnp.logical_and(t >= 0, t < cw))
        def _():
            pltpu.make_async_copy(tab_ref.at[isrc.at[s]], rows_bufs[s],
                                  g_sems[s]).wait()
            buf = rows_bufs[s]

            def row_body(r, cc):
                for col in range(comp_w // 16):
                    sl = pl.ds(col * 16, 16)
                    t0 = buf[r, sl]
                    buf[r, sl] = jnp.maximum(t0, 0.2 * t0)
                return cc

            lax.fori_loop(0, _B, row_body, 0, unroll=2)
            pltpu.async_copy(buf, shared.at[idst.at[s]], sc_sems[s],
                             add=True)

    ew_start(jnp.int32(0), 0)

    def group_body(u, carry):
        for si in range(3):
            t = 3 * u + si
            gather_start(t, si)
            ew_start(t + 1, (si + 1) % 3)
            compute(t - 1, (si + 2) % 3)
        return carry

    n_groups = (max_cw + 1 + 2) // 3
    lax.fori_loop(0, n_groups, group_body, 0)

    for s in range(3):
        pltpu.make_async_copy(rows_bufs[s], shared.at[idst.at[s]],
                              sc_sems[s]).wait()
    plsc.subcore_barrier()

    row0 = sid * rows_per_tile

    @pl.when(cid == 0)
    def _():
        pltpu.sync_copy(shared.at[pl.ds(row0, rows_per_tile)],
                        out0_ref.at[pl.ds(row0, rows_per_tile)])

    @pl.when(cid == 1)
    def _():
        pltpu.sync_copy(shared.at[pl.ds(row0, rows_per_tile)],
                        out1_ref.at[pl.ds(row0, rows_per_tile)])


def _sc_edge_body(*refs, n_pass, comp_ws, n_chunks, max_cw, rows_per_tile):
    tabs = refs[0:n_pass]
    ews = refs[n_pass:2 * n_pass]
    src_ref, dst_ref, zer_ref = refs[2 * n_pass:2 * n_pass + 3]
    outs = refs[2 * n_pass + 3:4 * n_pass + 3]
    (isrc, idst, rows0, rows1, rows2, shared, ew_sems, g_sems,
     sc_sems) = refs[4 * n_pass + 3:]
    cid = lax.axis_index("c")
    sid = lax.axis_index("s")
    wid = sid * _NC + cid
    lo = (n_chunks * wid) // _NW
    cw = (n_chunks * (wid + 1)) // _NW - lo
    for p in range(n_pass):
        _sc_one_pass(tabs[p], ews[p], src_ref, dst_ref, zer_ref,
                     outs[2 * p], outs[2 * p + 1],
                     isrc, idst, (rows0, rows1, rows2), shared,
                     ew_sems, g_sems, sc_sems, cid, sid, lo, cw,
                     comp_w=comp_ws[p], max_cw=max_cw,
                     rows_per_tile=rows_per_tile)


def _sc_edge_multi(tabs, ews, src, dst, comp_ws):
    n_pass = len(tabs)
    n_nodes, width = tabs[0].shape
    n_edges = src.shape[0]
    n_chunks = n_edges // _B
    max_cw = -(-n_chunks // _NW)
    rows_per_tile = -(-n_nodes // (8 * _NS)) * 8
    n_pad = rows_per_tile * _NS
    mesh = plsc.VectorSubcoreMesh(core_axis_name="c", subcore_axis_name="s",
                                  num_cores=_NC, num_subcores=_NS)
    body = functools.partial(
        _sc_edge_body, n_pass=n_pass, comp_ws=tuple(comp_ws),
        n_chunks=n_chunks, max_cw=max_cw, rows_per_tile=rows_per_tile)
    zer = jnp.zeros((rows_per_tile, width), jnp.float32)
    outs = pl.kernel(
        body,
        out_type=(jax.ShapeDtypeStruct((n_pad, width), jnp.float32),
                  ) * (2 * n_pass),
        mesh=mesh,
        scratch_types=[
            pltpu.VMEM((3, _B), jnp.int32),
            pltpu.VMEM((3, _B), jnp.int32),
            pltpu.VMEM((_B, width), jnp.float32),
            pltpu.VMEM((_B, width), jnp.float32),
            pltpu.VMEM((_B, width), jnp.float32),
            pltpu.VMEM_SHARED((n_pad, width), jnp.float32),
            [pltpu.SemaphoreType.DMA] * 3,
            [pltpu.SemaphoreType.DMA] * 3,
            [pltpu.SemaphoreType.DMA] * 3,
        ],
    )(*tabs, *ews, src, dst, zer)
    return outs


def _sc_edge(tab, ew, src, dst, comp_w=None):
    comp_w = tab.shape[1] if comp_w is None else comp_w
    return _sc_edge_multi([tab], [ew], src, dst, [comp_w])


def _layer_weights(p, din, hdim, de):
    w_x = p["msg_W"][:din]
    w_e = p["msg_W"][din:]
    if hdim < 128:
        padw = 128 - hdim
        tabw = 128
        tab_w = jnp.concatenate([w_x, jnp.zeros((din, padw))], axis=1)
        tab_b = jnp.zeros((tabw,), jnp.float32).at[hdim].set(1.0)
        ew_w = jnp.concatenate([w_e, jnp.zeros((de, padw))], axis=1)
        ew_b = jnp.concatenate([p["msg_b"], jnp.zeros((padw,), jnp.float32)])
    else:
        tabw = hdim
        tab_w = w_x
        tab_b = jnp.zeros((tabw,), jnp.float32)
        ew_w = w_e
        ew_b = p["msg_b"]
    cat_w = jnp.concatenate([tab_w, p["upd_W"][hdim:], p["res_W"]], axis=1)
    cat_b = jnp.concatenate([tab_b, p["upd_b"], p["res_b"]])
    tab_widths = [128] * (tabw // 128)
    return cat_w, cat_b, ew_w, ew_b, tab_widths


def kernel(x, edge_index, edge_attr, batch, params):
    src = edge_index[0]
    dst = edge_index[1]
    n, dn = x.shape
    de = edge_attr.shape[1]
    hid = (64, 128, 256)
    bm_n = 2000
    p0, p1, p2 = (params[f"layer{i}"] for i in range(3))

    cw0, cb0, eww0, ewb0, tw0 = _layer_weights(p0, dn, hid[0], de)
    cw1, cb1, eww1, ewb1, tw1 = _layer_weights(p1, hid[0], hid[1], de)
    cw2, cb2, eww2, ewb2, tw2 = _layer_weights(p2, hid[1], hid[2], de)

    ew_w = jnp.concatenate([eww0, eww1, eww2], axis=1)
    ew_b = jnp.concatenate([ewb0, ewb1, ewb2])
    ew0, ew1, ew2a, ew2b = _mm_multi(edge_attr, ew_w, ew_b,
                                     [128, 128, 128, 128], 6400)

    tab0, hu0, res0 = _mm_multi(x, cw0, cb0, [128, hid[0], hid[0]], bm_n)
    s0_0, s1_0 = _sc_edge(tab0, ew0, src, dst, comp_w=hid[0] + 16)
    c0 = s0_0[:, hid[0]:hid[0] + 16]
    c1 = s1_0[:, hid[0]:hid[0] + 16]

    tab1, hu1, res1 = _upd_mm(
        [s0_0, s1_0], None, hu0, res0, p0["upd_W"][:hid[0]], p0["ln_g"],
        p0["ln_b"], cw1, cb1, [128, hid[1], hid[1]], hid[0], bm_n)

    s0_1, s1_1 = _sc_edge(tab1, ew1, src, dst)

    tab2a, tab2b, hu2, res2 = _upd_mm(
        [s0_1, s1_1], (c0, c1), hu1, res1, p1["upd_W"][:hid[1]], p1["ln_g"],
        p1["ln_b"], cw2, cb2, [128, 128, hid[2], hid[2]], hid[1], bm_n)

    s0_2a, s1_2a, s0_2b, s1_2b = _sc_edge_multi(
        [tab2a, tab2b], [ew2a, ew2b], src, dst, [128, 128])

    batch2d = batch.reshape(n, 1)
    return _upd_head(
        [s0_2a, s0_2b, s1_2a, s1_2b], (c0, c1), hu2, res2,
        p2["upd_W"][:hid[2]], p2["ln_g"], p2["ln_b"], batch2d,
        params["mlp"], 64, bm_n)
